# Initial kernel scaffold; baseline (speedup 1.0000x reference)
import jax
import jax.numpy as jnp
from jax.experimental import pallas as pl


def kernel(x, edge_index, edge_attr, target, emb, core_W, core_b, filt_W, filt_b, bn1_g, bn1_b, bn2_g, bn2_b, fc1_W, fc1_b, fc2_W, fc2_b):
    raise NotImplementedError("write your pallas kernel here")


# speedup vs baseline: 4.9189x; 4.9189x over previous
import functools

import jax
import jax.numpy as jnp
from jax import lax
from jax.experimental import pallas as pl
from jax.experimental.pallas import tpu as pltpu
from jax.experimental.pallas import tpu_sc as plsc

f32 = jnp.float32
i32 = jnp.int32

_N = 50000
_E = 800000
_D = 64
_H = 32
_C = 128
_NSUB = 16
_EPAD = 802816
_EPT = _EPAD // _NSUB
_NCHUNK = _EPT // _C
_EPC = _EPAD // 2
_EPT2 = _EPC // _NSUB
_NCHUNK2 = _EPT2 // _C
_NPAD = 50176
_TPW = _NPAD // 32
_TCH = 112
_NTCH = _TPW // _TCH
_NSL = _N // _NSUB

_BLKN = 2000
_GN = _N // _BLKN
_BLKE = 1024
_GE = _EPAD // _BLKE
_BLKT = 512
_GT = _NPAD // _BLKT

_mesh = plsc.VectorSubcoreMesh(core_axis_name="c", subcore_axis_name="s",
                               num_cores=2, num_subcores=_NSUB)


def _passA_body(idxi, idxj, pil, pih, pjl, pjh, eal, eah, eaf, fi, fj, zn,
                hl, hh, ex, s_out, stats,
                fib, fjb, iv, jv, pib, pjb, eab, eafb, exb, statsb, s_sh,
                sem1, sem2):
    c = lax.axis_index("c")
    sid = lax.axis_index("s")
    base = sid * _EPT

    def half(pi_r, pj_r, ea_r, h_r, with_f):
        def chunk(i, carry):
            b0 = base + i * _C
            pltpu.sync_copy(idxi.at[pl.ds(b0, _C)], iv)
            pltpu.sync_copy(idxj.at[pl.ds(b0, _C)], jv)
            cp1 = pltpu.async_copy(pi_r.at[iv], pib, sem1)
            cp2 = pltpu.async_copy(pj_r.at[jv], pjb, sem2)
            pltpu.sync_copy(ea_r.at[pl.ds(b0, _C)], eab)
            cp1.wait()
            cp2.wait()

            def ebody(k, cr):
                sh0, sh1, sq0, sq1 = cr
                h0 = pib[k, pl.ds(0, 16)] + pjb[k, pl.ds(0, 16)] + eab[k, pl.ds(0, 16)]
                h1 = pib[k, pl.ds(16, 16)] + pjb[k, pl.ds(16, 16)] + eab[k, pl.ds(16, 16)]
                pib[k, pl.ds(0, 16)] = h0
                pib[k, pl.ds(16, 16)] = h1
                return (sh0 + h0, sh1 + h1, sq0 + h0 * h0, sq1 + h1 * h1)

            carry = lax.fori_loop(0, _C, ebody, carry)
            pltpu.sync_copy(pib, h_r.at[pl.ds(b0, _C)])
            if with_f:
                cp3 = pltpu.async_copy(fi.at[iv], fib, sem1)
                cp4 = pltpu.async_copy(fj.at[jv], fjb, sem2)
                pltpu.sync_copy(eaf.at[pl.ds(b0, _C)], eafb)
                cp3.wait()
                cp4.wait()
                for m in range(_C // 16):
                    sl = pl.ds(m * 16, 16)
                    f16 = fib[sl] + fjb[sl] + eafb[sl]
                    exb[sl] = jnp.exp(f16)
                pltpu.sync_copy(exb, ex.at[pl.ds(b0, _C)])
                pltpu.sync_copy(exb, s_sh.at[iv], add=True)
            return carry

        z = jnp.zeros((16,), f32)
        sh0, sh1, sq0, sq1 = lax.fori_loop(0, _NCHUNK, chunk, (z, z, z, z))
        statsb[0, pl.ds(0, 16)] = sh0
        statsb[0, pl.ds(16, 16)] = sh1
        statsb[1, pl.ds(0, 16)] = sq0
        statsb[1, pl.ds(16, 16)] = sq1
        pltpu.sync_copy(statsb, stats.at[c, sid])

    @pl.when(c == 0)
    def _():
        @pl.when(sid == 0)
        def _():
            pltpu.sync_copy(zn, s_sh)

        plsc.subcore_barrier()
        half(pil, pjl, eal, hl, True)
        plsc.subcore_barrier()

        @pl.when(sid == 0)
        def _():
            pltpu.sync_copy(s_sh, s_out)

    @pl.when(c == 1)
    def _():
        half(pih, pjh, eah, hh, False)


_passA = pl.kernel(
    _passA_body,
    out_type=[
        jax.ShapeDtypeStruct((_EPAD, _H), f32),
        jax.ShapeDtypeStruct((_EPAD, _H), f32),
        jax.ShapeDtypeStruct((_EPAD,), f32),
        jax.ShapeDtypeStruct((_N,), f32),
        jax.ShapeDtypeStruct((2, _NSUB, 2, _H), f32),
    ],
    mesh=_mesh,
    compiler_params=pltpu.CompilerParams(use_tc_tiling_on_sc=False),
    scratch_types=[
        pltpu.VMEM((_C,), f32),
        pltpu.VMEM((_C,), f32),
        pltpu.VMEM((_C,), i32),
        pltpu.VMEM((_C,), i32),
        pltpu.VMEM((_C, _H), f32),
        pltpu.VMEM((_C, _H), f32),
        pltpu.VMEM((_C, _H), f32),
        pltpu.VMEM((_C,), f32),
        pltpu.VMEM((_C,), f32),
        pltpu.VMEM((2, _H), f32),
        pltpu.VMEM_SHARED((_N,), f32),
        pltpu.SemaphoreType.DMA,
        pltpu.SemaphoreType.DMA,
    ],
)


def _passB_body(hl, hh, ex, idxi, ab, zn32, aggl, aggh,
                hb, exb, ib, abv, acc_sh, sem1):
    c = lax.axis_index("c")
    sid = lax.axis_index("s")
    base = sid * _EPT
    pltpu.sync_copy(ab.at[c], abv)

    @pl.when(sid == 0)
    def _():
        pltpu.sync_copy(zn32, acc_sh)

    plsc.subcore_barrier()
    a0 = abv[0, pl.ds(0, 16)]
    a1 = abv[0, pl.ds(16, 16)]
    b0v = abv[1, pl.ds(0, 16)]
    b1v = abv[1, pl.ds(16, 16)]

    def half(h_r, agg_r):
        def chunk(i, _):
            b0 = base + i * _C
            pltpu.sync_copy(h_r.at[pl.ds(b0, _C)], hb)
            pltpu.sync_copy(ex.at[pl.ds(b0, _C)], exb)
            pltpu.sync_copy(idxi.at[pl.ds(b0, _C)], ib)

            def gbody(g, __):
                ex16 = exb[pl.ds(g * 16, 16)]
                for j in range(16):
                    k = g * 16 + j
                    sc = ex16[j]
                    h0 = hb[k, pl.ds(0, 16)]
                    h1 = hb[k, pl.ds(16, 16)]
                    hb[k, pl.ds(0, 16)] = jnp.maximum(h0 * a0 + b0v, 0.0) * sc
                    hb[k, pl.ds(16, 16)] = jnp.maximum(h1 * a1 + b1v, 0.0) * sc
                return 0

            lax.fori_loop(0, _C // 16, gbody, 0)
            pltpu.sync_copy(hb, acc_sh.at[ib], add=True)
            return 0

        lax.fori_loop(0, _NCHUNK, chunk, 0)
        plsc.subcore_barrier()
        pltpu.sync_copy(acc_sh.at[pl.ds(sid * _NSL, _NSL)],
                        agg_r.at[pl.ds(sid * _NSL, _NSL)])

    @pl.when(c == 0)
    def _():
        half(hl, aggl)

    @pl.when(c == 1)
    def _():
        half(hh, aggh)


_passB = pl.kernel(
    _passB_body,
    out_type=[
        jax.ShapeDtypeStruct((_N, _H), f32),
        jax.ShapeDtypeStruct((_N, _H), f32),
    ],
    mesh=_mesh,
    compiler_params=pltpu.CompilerParams(use_tc_tiling_on_sc=False),
    scratch_types=[
        pltpu.VMEM((_C, _H), f32),
        pltpu.VMEM((_C,), f32),
        pltpu.VMEM((_C,), i32),
        pltpu.VMEM((2, _H), f32),
        pltpu.VMEM_SHARED((_N, _H), f32),
        pltpu.SemaphoreType.DMA,
    ],
)


def _cnt_body(idxi, zn, cnt2, ib, onesb, cnt_sh):
    c = lax.axis_index("c")
    sid = lax.axis_index("s")
    base = c * _EPC + sid * _EPT2

    @pl.when(sid == 0)
    def _():
        pltpu.sync_copy(zn, cnt_sh)

    plsc.subcore_barrier()
    lane = lax.iota(i32, 16)

    def chunk(i, _):
        b0 = base + i * _C
        pltpu.sync_copy(idxi.at[pl.ds(b0, _C)], ib)
        for m in range(_C // 16):
            gid = b0 + m * 16 + lane
            onesb[pl.ds(m * 16, 16)] = jnp.where(gid < _E, 1.0, 0.0)
        pltpu.sync_copy(onesb, cnt_sh.at[ib], add=True)
        return 0

    lax.fori_loop(0, _NCHUNK2, chunk, 0)
    plsc.subcore_barrier()

    @pl.when(sid == 0)
    def _():
        pltpu.sync_copy(cnt_sh, cnt2.at[c])


_cnt = pl.kernel(
    _cnt_body,
    out_type=jax.ShapeDtypeStruct((2, _N), f32),
    mesh=_mesh,
    compiler_params=pltpu.CompilerParams(use_tc_tiling_on_sc=False),
    scratch_types=[
        pltpu.VMEM((_C,), i32),
        pltpu.VMEM((_C,), f32),
        pltpu.VMEM_SHARED((_N,), f32),
    ],
)


def _gather_body(xf, tpad, crys, ib, rows, sem1):
    c = lax.axis_index("c")
    sid = lax.axis_index("s")
    w = sid * 2 + c
    base = w * _TPW

    def chunk(i, _):
        b0 = base + i * _TCH
        pltpu.sync_copy(tpad.at[pl.ds(b0, _TCH)], ib)
        pltpu.async_copy(xf.at[ib], rows, sem1).wait()
        pltpu.sync_copy(rows, crys.at[pl.ds(b0, _TCH)])
        return 0

    lax.fori_loop(0, _NTCH, chunk, 0)


_gather = pl.kernel(
    _gather_body,
    out_type=jax.ShapeDtypeStruct((_NPAD, _D), f32),
    mesh=_mesh,
    compiler_params=pltpu.CompilerParams(use_tc_tiling_on_sc=False),
    scratch_types=[
        pltpu.VMEM((_TCH,), i32),
        pltpu.VMEM((_TCH, _D), f32),
        pltpu.SemaphoreType.DMA,
    ],
)


def _embed_body(x_ref, emb_ref, out_ref):
    xr = x_ref[...]
    io = lax.broadcasted_iota(i32, (_BLKN, 128), 1)
    oh = (xr == io).astype(f32)
    out_ref[...] = jnp.dot(oh, emb_ref[...], preferred_element_type=f32)


_embed = pl.pallas_call(
    _embed_body,
    grid=(_GN,),
    in_specs=[
        pl.BlockSpec((_BLKN, 1), lambda i: (i, 0)),
        pl.BlockSpec((128, _D), lambda i: (0, 0)),
    ],
    out_specs=pl.BlockSpec((_BLKN, _D), lambda i: (i, 0)),
    out_shape=jax.ShapeDtypeStruct((_N, _D), f32),
)


def _nodeproj_body(xf_ref, wcat_ref, wf_ref, pil_ref, pih_ref, pjl_ref,
                   pjh_ref, fv_ref):
    acc = jnp.dot(xf_ref[...], wcat_ref[...], preferred_element_type=f32)
    pil_ref[...] = acc[:, 0:32]
    pih_ref[...] = acc[:, 32:64]
    pjl_ref[...] = acc[:, 64:96]
    pjh_ref[...] = acc[:, 96:128]
    fv_ref[...] = jnp.dot(xf_ref[...], wf_ref[...], preferred_element_type=f32)


_nodeproj = pl.pallas_call(
    _nodeproj_body,
    grid=(_GN,),
    in_specs=[
        pl.BlockSpec((_BLKN, _D), lambda i: (i, 0)),
        pl.BlockSpec((_D, 128), lambda i: (0, 0)),
        pl.BlockSpec((_D, 8), lambda i: (0, 0)),
    ],
    out_specs=[
        pl.BlockSpec((_BLKN, _H), lambda i: (i, 0)),
        pl.BlockSpec((_BLKN, _H), lambda i: (i, 0)),
        pl.BlockSpec((_BLKN, _H), lambda i: (i, 0)),
        pl.BlockSpec((_BLKN, _H), lambda i: (i, 0)),
        pl.BlockSpec((_BLKN, 8), lambda i: (i, 0)),
    ],
    out_shape=[
        jax.ShapeDtypeStruct((_N, _H), f32),
        jax.ShapeDtypeStruct((_N, _H), f32),
        jax.ShapeDtypeStruct((_N, _H), f32),
        jax.ShapeDtypeStruct((_N, _H), f32),
        jax.ShapeDtypeStruct((_N, 8), f32),
    ],
)


def _eaproj_body(ea_ref, w_ref, b_ref, eal_ref, eah_ref, eaf_ref):
    i = pl.program_id(0)
    acc = jnp.dot(ea_ref[...], w_ref[...], preferred_element_type=f32) + b_ref[...]
    rid = lax.broadcasted_iota(i32, (_BLKE, 1), 0) + i * _BLKE
    mask = rid < _E
    eal_ref[...] = jnp.where(mask, acc[:, 0:32], 0.0)
    eah_ref[...] = jnp.where(mask, acc[:, 32:64], 0.0)
    eaf_ref[...] = jnp.where(mask, acc[:, 64:72], -1e30)


_eaproj = pl.pallas_call(
    _eaproj_body,
    grid=(_GE,),
    in_specs=[
        pl.BlockSpec((_BLKE, 16), lambda i: (i, 0)),
        pl.BlockSpec((16, 128), lambda i: (0, 0)),
        pl.BlockSpec((1, 128), lambda i: (0, 0)),
    ],
    out_specs=[
        pl.BlockSpec((_BLKE, _H), lambda i: (i, 0)),
        pl.BlockSpec((_BLKE, _H), lambda i: (i, 0)),
        pl.BlockSpec((_BLKE, 8), lambda i: (i, 0)),
    ],
    out_shape=[
        jax.ShapeDtypeStruct((_EPAD, _H), f32),
        jax.ShapeDtypeStruct((_EPAD, _H), f32),
        jax.ShapeDtypeStruct((_EPAD, 8), f32),
    ],
)


def _k6a_body(aggl_ref, aggh_ref, s_ref, cnt_ref, agg_ref, ssum_ref, ssum2_ref):
    i = pl.program_id(0)
    scale = 1.0 / ((s_ref[...] + 1e-16) * jnp.maximum(cnt_ref[...], 1.0))
    a = jnp.concatenate([aggl_ref[...], aggh_ref[...]], axis=1) * scale
    agg_ref[...] = a

    @pl.when(i == 0)
    def _():
        ssum_ref[...] = jnp.zeros_like(ssum_ref)
        ssum2_ref[...] = jnp.zeros_like(ssum2_ref)

    ssum_ref[...] += jnp.sum(a, axis=0, keepdims=True)
    ssum2_ref[...] += jnp.sum(a * a, axis=0, keepdims=True)


_k6a = pl.pallas_call(
    _k6a_body,
    grid=(_GN,),
    in_specs=[
        pl.BlockSpec((_BLKN, _H), lambda i: (i, 0)),
        pl.BlockSpec((_BLKN, _H), lambda i: (i, 0)),
        pl.BlockSpec((_BLKN, 1), lambda i: (i, 0)),
        pl.BlockSpec((_BLKN, 1), lambda i: (i, 0)),
    ],
    out_specs=[
        pl.BlockSpec((_BLKN, _D), lambda i: (i, 0)),
        pl.BlockSpec((1, _D), lambda i: (0, 0)),
        pl.BlockSpec((1, _D), lambda i: (0, 0)),
    ],
    out_shape=[
        jax.ShapeDtypeStruct((_N, _D), f32),
        jax.ShapeDtypeStruct((1, _D), f32),
        jax.ShapeDtypeStruct((1, _D), f32),
    ],
)


def _k6b_body(agg_ref, xf_ref, a2_ref, b2_ref, out_ref):
    out_ref[...] = jnp.maximum(agg_ref[...] * a2_ref[...] + b2_ref[...]
                               + xf_ref[...], 0.0)


_k6b = pl.pallas_call(
    _k6b_body,
    grid=(_GN,),
    in_specs=[
        pl.BlockSpec((_BLKN, _D), lambda i: (i, 0)),
        pl.BlockSpec((_BLKN, _D), lambda i: (i, 0)),
        pl.BlockSpec((1, _D), lambda i: (0, 0)),
        pl.BlockSpec((1, _D), lambda i: (0, 0)),
    ],
    out_specs=pl.BlockSpec((_BLKN, _D), lambda i: (i, 0)),
    out_shape=jax.ShapeDtypeStruct((_N, _D), f32),
)


def _fc_body(cr_ref, w1_ref, b1_ref, dv_ref, db_ref, out_ref):
    c0 = jnp.maximum(cr_ref[...], 0.0)
    t1 = jnp.maximum(jnp.dot(c0, w1_ref[...], preferred_element_type=f32)
                     + b1_ref[...], 0.0)
    d8 = jnp.dot(t1, dv_ref[...], preferred_element_type=f32) + db_ref[...]
    d = d8[:, 0:1]
    p1 = 1.0 / (1.0 + jnp.exp(-d))
    out_ref[...] = jnp.concatenate([1.0 - p1, p1], axis=1)


_fc = pl.pallas_call(
    _fc_body,
    grid=(_GT,),
    in_specs=[
        pl.BlockSpec((_BLKT, _D), lambda i: (i, 0)),
        pl.BlockSpec((_D, _D), lambda i: (0, 0)),
        pl.BlockSpec((1, _D), lambda i: (0, 0)),
        pl.BlockSpec((_D, 8), lambda i: (0, 0)),
        pl.BlockSpec((1, 8), lambda i: (0, 0)),
    ],
    out_specs=pl.BlockSpec((_BLKT, 2), lambda i: (i, 0)),
    out_shape=jax.ShapeDtypeStruct((_NPAD, 2), f32),
)


def kernel(x, edge_index, edge_attr, target, emb, core_W, core_b, filt_W,
           filt_b, bn1_g, bn1_b, bn2_g, bn2_b, fc1_W, fc1_b, fc2_W, fc2_b):
    x = x.astype(i32)
    idx_i = edge_index[0].astype(i32)
    idx_j = edge_index[1].astype(i32)
    pad_e = _EPAD - _E
    idxi_p = jnp.concatenate([idx_i, jnp.zeros((pad_e,), i32)])
    idxj_p = jnp.concatenate([idx_j, jnp.zeros((pad_e,), i32)])
    ea_pad = jnp.concatenate([edge_attr.astype(f32),
                              jnp.zeros((pad_e, 16), f32)], axis=0)
    tpad = jnp.concatenate([target.astype(i32),
                            jnp.zeros((_NPAD - _N,), i32)])
    zn = jnp.zeros((_N,), f32)
    zn32 = jnp.zeros((_N, _H), f32)
    emb_pad = jnp.zeros((128, _D), f32).at[:100].set(emb)

    xf = _embed(x.reshape(_N, 1), emb_pad)
    cnt2 = _cnt(idxi_p, zn)
    cnt = (cnt2[0] + cnt2[1]).reshape(_N, 1)

    for l in range(3):
        cW = core_W[l]
        fW = filt_W[l]
        wcat = jnp.concatenate([cW[:_D, :_H], cW[:_D, _H:],
                                cW[_D:2 * _D, :_H], cW[_D:2 * _D, _H:]], axis=1)
        wf = jnp.zeros((_D, 8), f32).at[:, 0].set(fW[:_D, 0]) \
                                    .at[:, 1].set(fW[_D:2 * _D, 0])
        pil, pih, pjl, pjh, fv = _nodeproj(xf, wcat, wf)
        f_i = fv[:, 0]
        f_j = fv[:, 1]
        wec = jnp.zeros((16, 128), f32).at[:, :_D].set(cW[2 * _D:]) \
                                       .at[:, _D].set(fW[2 * _D:, 0])
        bec = jnp.zeros((1, 128), f32).at[0, :_D].set(core_b[l]) \
                                      .at[0, _D].set(filt_b[l][0])
        eal, eah, eaf8 = _eaproj(ea_pad, wec, bec)
        eaf = eaf8[:, 0]

        hl, hh, ex, s, stats = _passA(idxi_p, idxj_p, pil, pih, pjl, pjh,
                                      eal, eah, eaf, f_i, f_j, zn)

        sum_h = jnp.concatenate([stats[0, :, 0, :].sum(0),
                                 stats[1, :, 0, :].sum(0)])
        sum_h2 = jnp.concatenate([stats[0, :, 1, :].sum(0),
                                  stats[1, :, 1, :].sum(0)])
        hp = jnp.concatenate([pil[0] + pjl[0], pih[0] + pjh[0]])
        sum_h = sum_h - pad_e * hp
        sum_h2 = sum_h2 - pad_e * hp * hp
        mu = sum_h / _E
        var = sum_h2 / _E - mu * mu
        a_bn = bn1_g[l] * lax.rsqrt(var + 1e-5)
        b_bn = bn1_b[l] - mu * a_bn
        ab = jnp.stack([jnp.stack([a_bn[:_H], b_bn[:_H]]),
                        jnp.stack([a_bn[_H:], b_bn[_H:]])])

        aggl, aggh = _passB(hl, hh, ex, idxi_p, ab, zn32)

        agg, ssum, ssum2 = _k6a(aggl, aggh, s.reshape(_N, 1), cnt)
        mu2 = ssum / _N
        var2 = ssum2 / _N - mu2 * mu2
        a2 = bn2_g[l] * lax.rsqrt(var2 + 1e-5)
        b2 = bn2_b[l] - mu2 * a2
        xf = _k6b(agg, xf, a2.reshape(1, _D), b2.reshape(1, _D))

    crys = _gather(xf, tpad)
    dv = jnp.zeros((_D, 8), f32).at[:, 0].set(fc2_W[:, 1] - fc2_W[:, 0])
    db = jnp.zeros((1, 8), f32).at[0, 0].set(fc2_b[1] - fc2_b[0])
    out = _fc(crys, fc1_W, fc1_b.reshape(1, _D), dv, db)
    return out[:_N]


# speedup vs baseline: 6.5272x; 1.3270x over previous
import functools

import jax
import jax.numpy as jnp
from jax import lax
from jax.experimental import pallas as pl
from jax.experimental.pallas import tpu as pltpu
from jax.experimental.pallas import tpu_sc as plsc

f32 = jnp.float32
i32 = jnp.int32

_N = 50000
_E = 800000
_D = 64
_H = 32
_C = 128
_NSUB = 16
_EPAD = 802816
_EPT = _EPAD // _NSUB
_NCHUNK = _EPT // _C
_EPC = _EPAD // 2
_EPT2 = _EPC // _NSUB
_NCHUNK2 = _EPT2 // _C
_NPAD = 50176
_TPW = _NPAD // 32
_TCH = 112
_NTCH = _TPW // _TCH
_NSL = _N // _NSUB

_BLKN = 2000
_GN = _N // _BLKN
_BLKE = 1024
_GE = _EPAD // _BLKE
_BLKT = 512
_GT = _NPAD // _BLKT

_mesh = plsc.VectorSubcoreMesh(core_axis_name="c", subcore_axis_name="s",
                               num_cores=2, num_subcores=_NSUB)


_NG = _NCHUNK // 2


def _passA_body(idxi, idxj, pil, pih, pjl, pjh, eal, eah, eaf, fj, zn,
                hl, hh, ex, s2_out, stats,
                iv0, jv0, iv1, jv1, pib0, pjb0, eab0, pib1, pjb1, eab1,
                fjb, eafb, exb, statsb, s_sh,
                semA0, semA1, semW0, semW1):
    c = lax.axis_index("c")
    sid = lax.axis_index("s")
    base = sid * _EPT

    def run(pi_r, pj_r, ea_r, h_r, f_on_buf0):
        bufs = ((iv0, jv0, pib0, pjb0, eab0, semA0),
                (iv1, jv1, pib1, pjb1, eab1, semA1))
        with_f = (f_on_buf0, not f_on_buf0)

        def issue(cidx, b):
            iv, jv, pib, pjb, eab, sem = bufs[b]
            b0 = base + cidx * _C
            pltpu.sync_copy(idxi.at[pl.ds(b0, _C)], iv)
            pltpu.sync_copy(idxj.at[pl.ds(b0, _C)], jv)
            pltpu.async_copy(pi_r.at[iv], pib, sem)
            pltpu.async_copy(pj_r.at[jv], pjb, sem)
            pltpu.async_copy(ea_r.at[pl.ds(b0, _C)], eab, sem)
            if with_f[b]:
                pltpu.async_copy(fj.at[jv], fjb, sem)
                pltpu.async_copy(eaf.at[pl.ds(b0, _C)], eafb, sem)

        def wait_loads(cidx, b):
            iv, jv, pib, pjb, eab, sem = bufs[b]
            b0 = base + cidx * _C
            pltpu.make_async_copy(pi_r.at[iv], pib, sem).wait()
            pltpu.make_async_copy(pj_r.at[jv], pjb, sem).wait()
            pltpu.make_async_copy(ea_r.at[pl.ds(b0, _C)], eab, sem).wait()
            if with_f[b]:
                pltpu.make_async_copy(fj.at[jv], fjb, sem).wait()
                pltpu.make_async_copy(eaf.at[pl.ds(b0, _C)], eafb, sem).wait()

        def comp(cidx, b, carry):
            iv, jv, pib, pjb, eab, sem = bufs[b]

            def ebody(k, cr):
                sh0, sh1, sq0, sq1 = cr
                h0 = pib[k, pl.ds(0, 16)] + pjb[k, pl.ds(0, 16)] + eab[k, pl.ds(0, 16)]
                h1 = pib[k, pl.ds(16, 16)] + pjb[k, pl.ds(16, 16)] + eab[k, pl.ds(16, 16)]
                pib[k, pl.ds(0, 16)] = h0
                pib[k, pl.ds(16, 16)] = h1
                return (sh0 + h0, sh1 + h1, sq0 + h0 * h0, sq1 + h1 * h1)

            carry = lax.fori_loop(0, _C, ebody, carry, unroll=2)
            hwrite(cidx, b)
            if with_f[b]:
                for m in range(_C // 16):
                    sl = pl.ds(m * 16, 16)
                    exb[sl] = jnp.exp(fjb[sl] + eafb[sl])
                pltpu.sync_copy(exb, ex.at[pl.ds(base + cidx * _C, _C)])
                pltpu.sync_copy(exb, s_sh.at[iv], add=True)
            return carry

        def wsem(b):
            return semW0 if b == 0 else semW1

        def hwrite(cidx, b):
            pib = bufs[b][2]
            pltpu.async_copy(pib, h_r.at[pl.ds(base + cidx * _C, _C)], wsem(b))

        def hwait(cidx, b):
            pib = bufs[b][2]
            pltpu.make_async_copy(
                pib, h_r.at[pl.ds(base + cidx * _C, _C)], wsem(b)).wait()

        issue(0, 0)

        def body(gg, carry):
            c0 = 2 * gg
            c1 = c0 + 1

            @pl.when(gg > 0)
            def _():
                hwait(c1 - 2, 1)

            issue(c1, 1)
            wait_loads(c0, 0)
            carry = comp(c0, 0, carry)

            @pl.when(gg + 1 < _NG)
            def _():
                hwait(c0, 0)
                issue(c0 + 2, 0)

            wait_loads(c1, 1)
            carry = comp(c1, 1, carry)
            return carry

        z = jnp.zeros((16,), f32)
        sh0, sh1, sq0, sq1 = lax.fori_loop(0, _NG, body, (z, z, z, z))
        hwait(_NCHUNK - 2, 0)
        hwait(_NCHUNK - 1, 1)
        statsb[0, pl.ds(0, 16)] = sh0
        statsb[0, pl.ds(16, 16)] = sh1
        statsb[1, pl.ds(0, 16)] = sq0
        statsb[1, pl.ds(16, 16)] = sq1
        pltpu.sync_copy(statsb, stats.at[c, sid])

    @pl.when(sid == 0)
    def _():
        pltpu.sync_copy(zn, s_sh)

    plsc.subcore_barrier()

    @pl.when(c == 0)
    def _():
        run(pil, pjl, eal, hl, True)

    @pl.when(c == 1)
    def _():
        run(pih, pjh, eah, hh, False)

    plsc.subcore_barrier()

    @pl.when(sid == 0)
    def _():
        pltpu.sync_copy(s_sh, s2_out.at[c])


_passA = pl.kernel(
    _passA_body,
    out_type=[
        jax.ShapeDtypeStruct((_EPAD, _H), f32),
        jax.ShapeDtypeStruct((_EPAD, _H), f32),
        jax.ShapeDtypeStruct((_EPAD,), f32),
        jax.ShapeDtypeStruct((2, _N), f32),
        jax.ShapeDtypeStruct((2, _NSUB, 2, _H), f32),
    ],
    mesh=_mesh,
    compiler_params=pltpu.CompilerParams(use_tc_tiling_on_sc=False),
    scratch_types=[
        pltpu.VMEM((_C,), i32),
        pltpu.VMEM((_C,), i32),
        pltpu.VMEM((_C,), i32),
        pltpu.VMEM((_C,), i32),
        pltpu.VMEM((_C, _H), f32),
        pltpu.VMEM((_C, _H), f32),
        pltpu.VMEM((_C, _H), f32),
        pltpu.VMEM((_C, _H), f32),
        pltpu.VMEM((_C, _H), f32),
        pltpu.VMEM((_C, _H), f32),
        pltpu.VMEM((_C,), f32),
        pltpu.VMEM((_C,), f32),
        pltpu.VMEM((_C,), f32),
        pltpu.VMEM((2, _H), f32),
        pltpu.VMEM_SHARED((_N,), f32),
        pltpu.SemaphoreType.DMA,
        pltpu.SemaphoreType.DMA,
        pltpu.SemaphoreType.DMA,
        pltpu.SemaphoreType.DMA,
    ],
)


def _passB_body(hl, hh, ex, idxi, ab, zn32, aggl, aggh,
                hb0, exb0, ib0, hb1, exb1, ib1, abv, acc_sh, semB0, semB1):
    c = lax.axis_index("c")
    sid = lax.axis_index("s")
    base = sid * _EPT
    pltpu.sync_copy(ab.at[c], abv)

    @pl.when(sid == 0)
    def _():
        pltpu.sync_copy(zn32, acc_sh)

    plsc.subcore_barrier()
    a0 = abv[0, pl.ds(0, 16)]
    a1 = abv[0, pl.ds(16, 16)]
    b0v = abv[1, pl.ds(0, 16)]
    b1v = abv[1, pl.ds(16, 16)]

    def half(h_r, agg_r):
        bufs = ((hb0, exb0, ib0, semB0), (hb1, exb1, ib1, semB1))

        def issue(cidx, b):
            hb, exb, ib, sem = bufs[b]
            b0 = base + cidx * _C
            pltpu.async_copy(h_r.at[pl.ds(b0, _C)], hb, sem)
            pltpu.async_copy(ex.at[pl.ds(b0, _C)], exb, sem)
            pltpu.async_copy(idxi.at[pl.ds(b0, _C)], ib, sem)

        def wait_loads(cidx, b):
            hb, exb, ib, sem = bufs[b]
            b0 = base + cidx * _C
            pltpu.make_async_copy(h_r.at[pl.ds(b0, _C)], hb, sem).wait()
            pltpu.make_async_copy(ex.at[pl.ds(b0, _C)], exb, sem).wait()
            pltpu.make_async_copy(idxi.at[pl.ds(b0, _C)], ib, sem).wait()

        def comp(b):
            hb, exb, ib, sem = bufs[b]

            def gbody(g, __):
                ex16 = exb[pl.ds(g * 16, 16)]
                for j in range(16):
                    k = g * 16 + j
                    sc = ex16[j]
                    h0 = hb[k, pl.ds(0, 16)]
                    h1 = hb[k, pl.ds(16, 16)]
                    hb[k, pl.ds(0, 16)] = jnp.maximum(h0 * a0 + b0v, 0.0) * sc
                    hb[k, pl.ds(16, 16)] = jnp.maximum(h1 * a1 + b1v, 0.0) * sc
                return 0

            lax.fori_loop(0, _C // 16, gbody, 0)
            pltpu.sync_copy(hb, acc_sh.at[ib], add=True)

        issue(0, 0)

        def body(gg, _):
            c0 = 2 * gg
            issue(c0 + 1, 1)
            wait_loads(c0, 0)
            comp(0)

            @pl.when(gg + 1 < _NG)
            def _():
                issue(c0 + 2, 0)

            wait_loads(c0 + 1, 1)
            comp(1)
            return 0

        lax.fori_loop(0, _NG, body, 0)
        plsc.subcore_barrier()
        pltpu.sync_copy(acc_sh.at[pl.ds(sid * _NSL, _NSL)],
                        agg_r.at[pl.ds(sid * _NSL, _NSL)])

    @pl.when(c == 0)
    def _():
        half(hl, aggl)

    @pl.when(c == 1)
    def _():
        half(hh, aggh)


_passB = pl.kernel(
    _passB_body,
    out_type=[
        jax.ShapeDtypeStruct((_N, _H), f32),
        jax.ShapeDtypeStruct((_N, _H), f32),
    ],
    mesh=_mesh,
    compiler_params=pltpu.CompilerParams(use_tc_tiling_on_sc=False),
    scratch_types=[
        pltpu.VMEM((_C, _H), f32),
        pltpu.VMEM((_C,), f32),
        pltpu.VMEM((_C,), i32),
        pltpu.VMEM((_C, _H), f32),
        pltpu.VMEM((_C,), f32),
        pltpu.VMEM((_C,), i32),
        pltpu.VMEM((2, _H), f32),
        pltpu.VMEM_SHARED((_N, _H), f32),
        pltpu.SemaphoreType.DMA,
        pltpu.SemaphoreType.DMA,
    ],
)


def _cnt_body(idxi, zn, cnt2, ib, onesb, cnt_sh):
    c = lax.axis_index("c")
    sid = lax.axis_index("s")
    base = c * _EPC + sid * _EPT2

    @pl.when(sid == 0)
    def _():
        pltpu.sync_copy(zn, cnt_sh)

    plsc.subcore_barrier()
    lane = lax.iota(i32, 16)

    def chunk(i, _):
        b0 = base + i * _C
        pltpu.sync_copy(idxi.at[pl.ds(b0, _C)], ib)
        for m in range(_C // 16):
            gid = b0 + m * 16 + lane
            onesb[pl.ds(m * 16, 16)] = jnp.where(gid < _E, 1.0, 0.0)
        pltpu.sync_copy(onesb, cnt_sh.at[ib], add=True)
        return 0

    lax.fori_loop(0, _NCHUNK2, chunk, 0)
    plsc.subcore_barrier()

    @pl.when(sid == 0)
    def _():
        pltpu.sync_copy(cnt_sh, cnt2.at[c])


_cnt = pl.kernel(
    _cnt_body,
    out_type=jax.ShapeDtypeStruct((2, _N), f32),
    mesh=_mesh,
    compiler_params=pltpu.CompilerParams(use_tc_tiling_on_sc=False),
    scratch_types=[
        pltpu.VMEM((_C,), i32),
        pltpu.VMEM((_C,), f32),
        pltpu.VMEM_SHARED((_N,), f32),
    ],
)


def _gather_body(xf, tpad, crys, ib, rows, sem1):
    c = lax.axis_index("c")
    sid = lax.axis_index("s")
    w = sid * 2 + c
    base = w * _TPW

    def chunk(i, _):
        b0 = base + i * _TCH
        pltpu.sync_copy(tpad.at[pl.ds(b0, _TCH)], ib)
        pltpu.async_copy(xf.at[ib], rows, sem1).wait()
        pltpu.sync_copy(rows, crys.at[pl.ds(b0, _TCH)])
        return 0

    lax.fori_loop(0, _NTCH, chunk, 0)


_gather = pl.kernel(
    _gather_body,
    out_type=jax.ShapeDtypeStruct((_NPAD, _D), f32),
    mesh=_mesh,
    compiler_params=pltpu.CompilerParams(use_tc_tiling_on_sc=False),
    scratch_types=[
        pltpu.VMEM((_TCH,), i32),
        pltpu.VMEM((_TCH, _D), f32),
        pltpu.SemaphoreType.DMA,
    ],
)


def _embed_body(x_ref, emb_ref, out_ref):
    xr = x_ref[...]
    io = lax.broadcasted_iota(i32, (_BLKN, 128), 1)
    oh = (xr == io).astype(f32)
    out_ref[...] = jnp.dot(oh, emb_ref[...], preferred_element_type=f32)


_embed = pl.pallas_call(
    _embed_body,
    grid=(_GN,),
    in_specs=[
        pl.BlockSpec((_BLKN, 1), lambda i: (i, 0)),
        pl.BlockSpec((128, _D), lambda i: (0, 0)),
    ],
    out_specs=pl.BlockSpec((_BLKN, _D), lambda i: (i, 0)),
    out_shape=jax.ShapeDtypeStruct((_N, _D), f32),
)


def _nodeproj_body(xf_ref, wcat_ref, wf_ref, pil_ref, pih_ref, pjl_ref,
                   pjh_ref, fv_ref):
    acc = jnp.dot(xf_ref[...], wcat_ref[...], preferred_element_type=f32)
    pil_ref[...] = acc[:, 0:32]
    pih_ref[...] = acc[:, 32:64]
    pjl_ref[...] = acc[:, 64:96]
    pjh_ref[...] = acc[:, 96:128]
    fv_ref[...] = jnp.dot(xf_ref[...], wf_ref[...], preferred_element_type=f32)


_nodeproj = pl.pallas_call(
    _nodeproj_body,
    grid=(_GN,),
    in_specs=[
        pl.BlockSpec((_BLKN, _D), lambda i: (i, 0)),
        pl.BlockSpec((_D, 128), lambda i: (0, 0)),
        pl.BlockSpec((_D, 8), lambda i: (0, 0)),
    ],
    out_specs=[
        pl.BlockSpec((_BLKN, _H), lambda i: (i, 0)),
        pl.BlockSpec((_BLKN, _H), lambda i: (i, 0)),
        pl.BlockSpec((_BLKN, _H), lambda i: (i, 0)),
        pl.BlockSpec((_BLKN, _H), lambda i: (i, 0)),
        pl.BlockSpec((_BLKN, 8), lambda i: (i, 0)),
    ],
    out_shape=[
        jax.ShapeDtypeStruct((_N, _H), f32),
        jax.ShapeDtypeStruct((_N, _H), f32),
        jax.ShapeDtypeStruct((_N, _H), f32),
        jax.ShapeDtypeStruct((_N, _H), f32),
        jax.ShapeDtypeStruct((_N, 8), f32),
    ],
)


def _eaproj_body(ea_ref, w_ref, b_ref, eal_ref, eah_ref, eaf_ref):
    i = pl.program_id(0)
    acc = jnp.dot(ea_ref[...], w_ref[...], preferred_element_type=f32) + b_ref[...]
    rid = lax.broadcasted_iota(i32, (_BLKE, 1), 0) + i * _BLKE
    mask = rid < _E
    eal_ref[...] = jnp.where(mask, acc[:, 0:32], 0.0)
    eah_ref[...] = jnp.where(mask, acc[:, 32:64], 0.0)
    eaf_ref[...] = jnp.where(mask, acc[:, 64:72], -1e30)


_eaproj = pl.pallas_call(
    _eaproj_body,
    grid=(_GE,),
    in_specs=[
        pl.BlockSpec((_BLKE, 16), lambda i: (i, 0)),
        pl.BlockSpec((16, 128), lambda i: (0, 0)),
        pl.BlockSpec((1, 128), lambda i: (0, 0)),
    ],
    out_specs=[
        pl.BlockSpec((_BLKE, _H), lambda i: (i, 0)),
        pl.BlockSpec((_BLKE, _H), lambda i: (i, 0)),
        pl.BlockSpec((_BLKE, 8), lambda i: (i, 0)),
    ],
    out_shape=[
        jax.ShapeDtypeStruct((_EPAD, _H), f32),
        jax.ShapeDtypeStruct((_EPAD, _H), f32),
        jax.ShapeDtypeStruct((_EPAD, 8), f32),
    ],
)


def _k6a_body(aggl_ref, aggh_ref, s_ref, cnt_ref, agg_ref, ssum_ref, ssum2_ref):
    i = pl.program_id(0)
    scale = 1.0 / ((s_ref[...] + 1e-16) * jnp.maximum(cnt_ref[...], 1.0))
    a = jnp.concatenate([aggl_ref[...], aggh_ref[...]], axis=1) * scale
    agg_ref[...] = a

    @pl.when(i == 0)
    def _():
        ssum_ref[...] = jnp.zeros_like(ssum_ref)
        ssum2_ref[...] = jnp.zeros_like(ssum2_ref)

    ssum_ref[...] += jnp.sum(a, axis=0, keepdims=True)
    ssum2_ref[...] += jnp.sum(a * a, axis=0, keepdims=True)


_k6a = pl.pallas_call(
    _k6a_body,
    grid=(_GN,),
    in_specs=[
        pl.BlockSpec((_BLKN, _H), lambda i: (i, 0)),
        pl.BlockSpec((_BLKN, _H), lambda i: (i, 0)),
        pl.BlockSpec((_BLKN, 1), lambda i: (i, 0)),
        pl.BlockSpec((_BLKN, 1), lambda i: (i, 0)),
    ],
    out_specs=[
        pl.BlockSpec((_BLKN, _D), lambda i: (i, 0)),
        pl.BlockSpec((1, _D), lambda i: (0, 0)),
        pl.BlockSpec((1, _D), lambda i: (0, 0)),
    ],
    out_shape=[
        jax.ShapeDtypeStruct((_N, _D), f32),
        jax.ShapeDtypeStruct((1, _D), f32),
        jax.ShapeDtypeStruct((1, _D), f32),
    ],
)


def _k6b_body(agg_ref, xf_ref, a2_ref, b2_ref, out_ref):
    out_ref[...] = jnp.maximum(agg_ref[...] * a2_ref[...] + b2_ref[...]
                               + xf_ref[...], 0.0)


_k6b = pl.pallas_call(
    _k6b_body,
    grid=(_GN,),
    in_specs=[
        pl.BlockSpec((_BLKN, _D), lambda i: (i, 0)),
        pl.BlockSpec((_BLKN, _D), lambda i: (i, 0)),
        pl.BlockSpec((1, _D), lambda i: (0, 0)),
        pl.BlockSpec((1, _D), lambda i: (0, 0)),
    ],
    out_specs=pl.BlockSpec((_BLKN, _D), lambda i: (i, 0)),
    out_shape=jax.ShapeDtypeStruct((_N, _D), f32),
)


def _fc_body(cr_ref, w1_ref, b1_ref, dv_ref, db_ref, out_ref):
    c0 = jnp.maximum(cr_ref[...], 0.0)
    t1 = jnp.maximum(jnp.dot(c0, w1_ref[...], preferred_element_type=f32)
                     + b1_ref[...], 0.0)
    d8 = jnp.dot(t1, dv_ref[...], preferred_element_type=f32) + db_ref[...]
    d = d8[:, 0:1]
    p1 = 1.0 / (1.0 + jnp.exp(-d))
    out_ref[...] = jnp.concatenate([1.0 - p1, p1], axis=1)


_fc = pl.pallas_call(
    _fc_body,
    grid=(_GT,),
    in_specs=[
        pl.BlockSpec((_BLKT, _D), lambda i: (i, 0)),
        pl.BlockSpec((_D, _D), lambda i: (0, 0)),
        pl.BlockSpec((1, _D), lambda i: (0, 0)),
        pl.BlockSpec((_D, 8), lambda i: (0, 0)),
        pl.BlockSpec((1, 8), lambda i: (0, 0)),
    ],
    out_specs=pl.BlockSpec((_BLKT, 2), lambda i: (i, 0)),
    out_shape=jax.ShapeDtypeStruct((_NPAD, 2), f32),
)


def kernel(x, edge_index, edge_attr, target, emb, core_W, core_b, filt_W,
           filt_b, bn1_g, bn1_b, bn2_g, bn2_b, fc1_W, fc1_b, fc2_W, fc2_b):
    x = x.astype(i32)
    idx_i = edge_index[0].astype(i32)
    idx_j = edge_index[1].astype(i32)
    pad_e = _EPAD - _E
    idxi_p = jnp.concatenate([idx_i, jnp.zeros((pad_e,), i32)])
    idxj_p = jnp.concatenate([idx_j, jnp.zeros((pad_e,), i32)])
    ea_pad = jnp.concatenate([edge_attr.astype(f32),
                              jnp.zeros((pad_e, 16), f32)], axis=0)
    tpad = jnp.concatenate([target.astype(i32),
                            jnp.zeros((_NPAD - _N,), i32)])
    zn = jnp.zeros((_N,), f32)
    zn32 = jnp.zeros((_N, _H), f32)
    emb_pad = jnp.zeros((128, _D), f32).at[:100].set(emb)

    xf = _embed(x.reshape(_N, 1), emb_pad)
    cnt2 = _cnt(idxi_p, zn)
    cnt = (cnt2[0] + cnt2[1]).reshape(_N, 1)

    for l in range(3):
        cW = core_W[l]
        fW = filt_W[l]
        wcat = jnp.concatenate([cW[:_D, :_H], cW[:_D, _H:],
                                cW[_D:2 * _D, :_H], cW[_D:2 * _D, _H:]], axis=1)
        wf = jnp.zeros((_D, 8), f32).at[:, 0].set(fW[_D:2 * _D, 0])
        pil, pih, pjl, pjh, fv = _nodeproj(xf, wcat, wf)
        f_j = fv[:, 0]
        wec = jnp.zeros((16, 128), f32).at[:, :_D].set(cW[2 * _D:]) \
                                       .at[:, _D].set(fW[2 * _D:, 0])
        bec = jnp.zeros((1, 128), f32).at[0, :_D].set(core_b[l]) \
                                      .at[0, _D].set(filt_b[l][0])
        eal, eah, eaf8 = _eaproj(ea_pad, wec, bec)
        eaf = eaf8[:, 0]

        hl, hh, ex, s2, stats = _passA(idxi_p, idxj_p, pil, pih, pjl, pjh,
                                       eal, eah, eaf, f_j, zn)
        s = s2[0] + s2[1]

        sum_h = jnp.concatenate([stats[0, :, 0, :].sum(0),
                                 stats[1, :, 0, :].sum(0)])
        sum_h2 = jnp.concatenate([stats[0, :, 1, :].sum(0),
                                  stats[1, :, 1, :].sum(0)])
        hp = jnp.concatenate([pil[0] + pjl[0], pih[0] + pjh[0]])
        sum_h = sum_h - pad_e * hp
        sum_h2 = sum_h2 - pad_e * hp * hp
        mu = sum_h / _E
        var = sum_h2 / _E - mu * mu
        a_bn = bn1_g[l] * lax.rsqrt(var + 1e-5)
        b_bn = bn1_b[l] - mu * a_bn
        ab = jnp.stack([jnp.stack([a_bn[:_H], b_bn[:_H]]),
                        jnp.stack([a_bn[_H:], b_bn[_H:]])])

        aggl, aggh = _passB(hl, hh, ex, idxi_p, ab, zn32)

        agg, ssum, ssum2 = _k6a(aggl, aggh, s.reshape(_N, 1), cnt)
        mu2 = ssum / _N
        var2 = ssum2 / _N - mu2 * mu2
        a2 = bn2_g[l] * lax.rsqrt(var2 + 1e-5)
        b2 = bn2_b[l] - mu2 * a2
        xf = _k6b(agg, xf, a2.reshape(1, _D), b2.reshape(1, _D))

    crys = _gather(xf, tpad)
    dv = jnp.zeros((_D, 8), f32).at[:, 0].set(fc2_W[:, 1] - fc2_W[:, 0])
    db = jnp.zeros((1, 8), f32).at[0, 0].set(fc2_b[1] - fc2_b[0])
    out = _fc(crys, fc1_W, fc1_b.reshape(1, _D), dv, db)
    return out[:_N]


# speedup vs baseline: 7.2631x; 1.1127x over previous
import functools

import jax
import jax.numpy as jnp
from jax import lax
from jax.experimental import pallas as pl
from jax.experimental.pallas import tpu as pltpu
from jax.experimental.pallas import tpu_sc as plsc

f32 = jnp.float32
i32 = jnp.int32

_N = 50000
_E = 800000
_D = 64
_H = 32
_C = 128
_NSUB = 16
_EPAD = 802816
_EPT = _EPAD // _NSUB
_NCHUNK = _EPT // _C
_EPC = _EPAD // 2
_EPT2 = _EPC // _NSUB
_NCHUNK2 = _EPT2 // _C
_NPAD = 50176
_TPW = _NPAD // 32
_TCH = 112
_NTCH = _TPW // _TCH
_NSL = _N // _NSUB

_BLKN = 2000
_GN = _N // _BLKN
_BLKE = 4096
_GE = _EPAD // _BLKE
_BLKT = 1792
_GT = _NPAD // _BLKT

_mesh = plsc.VectorSubcoreMesh(core_axis_name="c", subcore_axis_name="s",
                               num_cores=2, num_subcores=_NSUB)


_NG = _NCHUNK // 2


def _passA_body(do_cnt, *refs):
    if do_cnt:
        (idxi, idxj, pil, pih, pjl, pjh, eal, eah, eaf, fj, zn,
         hl, hh, ex, s2_out, stats, cnt2,
         iv0, jv0, iv1, jv1, pib0, pjb0, eab0, pib1, pjb1, eab1,
         fjb, eafb, exb, onesb, statsb, s_sh, cnt_sh,
         semA0, semA1, semW0, semW1) = refs
    else:
        (idxi, idxj, pil, pih, pjl, pjh, eal, eah, eaf, fj, zn,
         hl, hh, ex, s2_out, stats,
         iv0, jv0, iv1, jv1, pib0, pjb0, eab0, pib1, pjb1, eab1,
         fjb, eafb, exb, onesb, statsb, s_sh, cnt_sh,
         semA0, semA1, semW0, semW1) = refs
    c = lax.axis_index("c")
    sid = lax.axis_index("s")
    base = sid * _EPT
    lane = lax.iota(i32, 16)

    def run(pi_r, pj_r, ea_r, h_r, f_on_buf0):
        bufs = ((iv0, jv0, pib0, pjb0, eab0, semA0),
                (iv1, jv1, pib1, pjb1, eab1, semA1))
        with_f = (f_on_buf0, not f_on_buf0)

        def issue(cidx, b):
            iv, jv, pib, pjb, eab, sem = bufs[b]
            b0 = base + cidx * _C
            pltpu.sync_copy(idxi.at[pl.ds(b0, _C)], iv)
            pltpu.sync_copy(idxj.at[pl.ds(b0, _C)], jv)
            pltpu.async_copy(pi_r.at[iv], pib, sem)
            pltpu.async_copy(pj_r.at[jv], pjb, sem)
            pltpu.async_copy(ea_r.at[pl.ds(b0, _C)], eab, sem)
            if with_f[b]:
                pltpu.async_copy(fj.at[jv], fjb, sem)
                pltpu.async_copy(eaf.at[pl.ds(b0, _C)], eafb, sem)

        def wait_loads(cidx, b):
            iv, jv, pib, pjb, eab, sem = bufs[b]
            b0 = base + cidx * _C
            pltpu.make_async_copy(pi_r.at[iv], pib, sem).wait()
            pltpu.make_async_copy(pj_r.at[jv], pjb, sem).wait()
            pltpu.make_async_copy(ea_r.at[pl.ds(b0, _C)], eab, sem).wait()
            if with_f[b]:
                pltpu.make_async_copy(fj.at[jv], fjb, sem).wait()
                pltpu.make_async_copy(eaf.at[pl.ds(b0, _C)], eafb, sem).wait()

        def comp(cidx, b, carry):
            iv, jv, pib, pjb, eab, sem = bufs[b]

            def ebody(k, cr):
                sh0, sh1, sq0, sq1 = cr
                h0 = pib[k, pl.ds(0, 16)] + pjb[k, pl.ds(0, 16)] + eab[k, pl.ds(0, 16)]
                h1 = pib[k, pl.ds(16, 16)] + pjb[k, pl.ds(16, 16)] + eab[k, pl.ds(16, 16)]
                pib[k, pl.ds(0, 16)] = h0
                pib[k, pl.ds(16, 16)] = h1
                return (sh0 + h0, sh1 + h1, sq0 + h0 * h0, sq1 + h1 * h1)

            carry = lax.fori_loop(0, _C, ebody, carry, unroll=2)
            hwrite(cidx, b)
            if with_f[b]:
                for m in range(_C // 16):
                    sl = pl.ds(m * 16, 16)
                    exb[sl] = jnp.exp(fjb[sl] + eafb[sl])
                pltpu.sync_copy(exb, ex.at[pl.ds(base + cidx * _C, _C)])
                pltpu.sync_copy(exb, s_sh.at[iv], add=True)
                if do_cnt:
                    b0 = base + cidx * _C
                    for m in range(_C // 16):
                        gid = b0 + m * 16 + lane
                        onesb[pl.ds(m * 16, 16)] = jnp.where(gid < _E, 1.0, 0.0)
                    pltpu.sync_copy(onesb, cnt_sh.at[iv], add=True)
            return carry

        def wsem(b):
            return semW0 if b == 0 else semW1

        def hwrite(cidx, b):
            pib = bufs[b][2]
            pltpu.async_copy(pib, h_r.at[pl.ds(base + cidx * _C, _C)], wsem(b))

        def hwait(cidx, b):
            pib = bufs[b][2]
            pltpu.make_async_copy(
                pib, h_r.at[pl.ds(base + cidx * _C, _C)], wsem(b)).wait()

        issue(0, 0)

        def body(gg, carry):
            c0 = 2 * gg
            c1 = c0 + 1

            @pl.when(gg > 0)
            def _():
                hwait(c1 - 2, 1)

            issue(c1, 1)
            wait_loads(c0, 0)
            carry = comp(c0, 0, carry)

            @pl.when(gg + 1 < _NG)
            def _():
                hwait(c0, 0)
                issue(c0 + 2, 0)

            wait_loads(c1, 1)
            carry = comp(c1, 1, carry)
            return carry

        z = jnp.zeros((16,), f32)
        sh0, sh1, sq0, sq1 = lax.fori_loop(0, _NG, body, (z, z, z, z))
        hwait(_NCHUNK - 2, 0)
        hwait(_NCHUNK - 1, 1)
        statsb[0, pl.ds(0, 16)] = sh0
        statsb[0, pl.ds(16, 16)] = sh1
        statsb[1, pl.ds(0, 16)] = sq0
        statsb[1, pl.ds(16, 16)] = sq1
        pltpu.sync_copy(statsb, stats.at[c, sid])

    @pl.when(sid == 0)
    def _():
        pltpu.sync_copy(zn, s_sh)
        if do_cnt:
            pltpu.sync_copy(zn, cnt_sh)

    plsc.subcore_barrier()

    @pl.when(c == 0)
    def _():
        run(pil, pjl, eal, hl, True)

    @pl.when(c == 1)
    def _():
        run(pih, pjh, eah, hh, False)

    plsc.subcore_barrier()

    @pl.when(sid == 0)
    def _():
        pltpu.sync_copy(s_sh, s2_out.at[c])
        if do_cnt:
            pltpu.sync_copy(cnt_sh, cnt2.at[c])


_passA_outs = [
    jax.ShapeDtypeStruct((_EPAD, _H), f32),
    jax.ShapeDtypeStruct((_EPAD, _H), f32),
    jax.ShapeDtypeStruct((_EPAD,), f32),
    jax.ShapeDtypeStruct((2, _N), f32),
    jax.ShapeDtypeStruct((2, _NSUB, 2, _H), f32),
]
_passA_scratch = [
        pltpu.VMEM((_C,), i32),
        pltpu.VMEM((_C,), i32),
        pltpu.VMEM((_C,), i32),
        pltpu.VMEM((_C,), i32),
        pltpu.VMEM((_C, _H), f32),
        pltpu.VMEM((_C, _H), f32),
        pltpu.VMEM((_C, _H), f32),
        pltpu.VMEM((_C, _H), f32),
        pltpu.VMEM((_C, _H), f32),
        pltpu.VMEM((_C, _H), f32),
        pltpu.VMEM((_C,), f32),
        pltpu.VMEM((_C,), f32),
        pltpu.VMEM((_C,), f32),
        pltpu.VMEM((_C,), f32),
        pltpu.VMEM((2, _H), f32),
        pltpu.VMEM_SHARED((_N,), f32),
        pltpu.VMEM_SHARED((_N,), f32),
        pltpu.SemaphoreType.DMA,
        pltpu.SemaphoreType.DMA,
        pltpu.SemaphoreType.DMA,
        pltpu.SemaphoreType.DMA,
]

_passA = pl.kernel(
    functools.partial(_passA_body, False),
    out_type=_passA_outs,
    mesh=_mesh,
    compiler_params=pltpu.CompilerParams(use_tc_tiling_on_sc=False),
    scratch_types=_passA_scratch,
)

_passA0 = pl.kernel(
    functools.partial(_passA_body, True),
    out_type=_passA_outs + [jax.ShapeDtypeStruct((2, _N), f32)],
    mesh=_mesh,
    compiler_params=pltpu.CompilerParams(use_tc_tiling_on_sc=False),
    scratch_types=_passA_scratch,
)


def _passB_body(hl, hh, ex, idxi, ab, zn32, aggl, aggh,
                hb0, exb0, ib0, hb1, exb1, ib1, abv, acc_sh, semB0, semB1):
    c = lax.axis_index("c")
    sid = lax.axis_index("s")
    base = sid * _EPT
    pltpu.sync_copy(ab.at[c], abv)

    @pl.when(sid == 0)
    def _():
        pltpu.sync_copy(zn32, acc_sh)

    plsc.subcore_barrier()
    a0 = abv[0, pl.ds(0, 16)]
    a1 = abv[0, pl.ds(16, 16)]
    b0v = abv[1, pl.ds(0, 16)]
    b1v = abv[1, pl.ds(16, 16)]

    def half(h_r, agg_r):
        bufs = ((hb0, exb0, ib0, semB0), (hb1, exb1, ib1, semB1))

        def issue(cidx, b):
            hb, exb, ib, sem = bufs[b]
            b0 = base + cidx * _C
            pltpu.async_copy(h_r.at[pl.ds(b0, _C)], hb, sem)
            pltpu.async_copy(ex.at[pl.ds(b0, _C)], exb, sem)
            pltpu.async_copy(idxi.at[pl.ds(b0, _C)], ib, sem)

        def wait_loads(cidx, b):
            hb, exb, ib, sem = bufs[b]
            b0 = base + cidx * _C
            pltpu.make_async_copy(h_r.at[pl.ds(b0, _C)], hb, sem).wait()
            pltpu.make_async_copy(ex.at[pl.ds(b0, _C)], exb, sem).wait()
            pltpu.make_async_copy(idxi.at[pl.ds(b0, _C)], ib, sem).wait()

        def comp(b):
            hb, exb, ib, sem = bufs[b]

            def gbody(g, __):
                ex16 = exb[pl.ds(g * 16, 16)]
                for j in range(16):
                    k = g * 16 + j
                    sc = ex16[j]
                    h0 = hb[k, pl.ds(0, 16)]
                    h1 = hb[k, pl.ds(16, 16)]
                    hb[k, pl.ds(0, 16)] = jnp.maximum(h0 * a0 + b0v, 0.0) * sc
                    hb[k, pl.ds(16, 16)] = jnp.maximum(h1 * a1 + b1v, 0.0) * sc
                return 0

            lax.fori_loop(0, _C // 16, gbody, 0)
            pltpu.sync_copy(hb, acc_sh.at[ib], add=True)

        issue(0, 0)

        def body(gg, _):
            c0 = 2 * gg
            issue(c0 + 1, 1)
            wait_loads(c0, 0)
            comp(0)

            @pl.when(gg + 1 < _NG)
            def _():
                issue(c0 + 2, 0)

            wait_loads(c0 + 1, 1)
            comp(1)
            return 0

        lax.fori_loop(0, _NG, body, 0)
        plsc.subcore_barrier()
        pltpu.sync_copy(acc_sh.at[pl.ds(sid * _NSL, _NSL)],
                        agg_r.at[pl.ds(sid * _NSL, _NSL)])

    @pl.when(c == 0)
    def _():
        half(hl, aggl)

    @pl.when(c == 1)
    def _():
        half(hh, aggh)


_passB = pl.kernel(
    _passB_body,
    out_type=[
        jax.ShapeDtypeStruct((_N, _H), f32),
        jax.ShapeDtypeStruct((_N, _H), f32),
    ],
    mesh=_mesh,
    compiler_params=pltpu.CompilerParams(use_tc_tiling_on_sc=False),
    scratch_types=[
        pltpu.VMEM((_C, _H), f32),
        pltpu.VMEM((_C,), f32),
        pltpu.VMEM((_C,), i32),
        pltpu.VMEM((_C, _H), f32),
        pltpu.VMEM((_C,), f32),
        pltpu.VMEM((_C,), i32),
        pltpu.VMEM((2, _H), f32),
        pltpu.VMEM_SHARED((_N, _H), f32),
        pltpu.SemaphoreType.DMA,
        pltpu.SemaphoreType.DMA,
    ],
)


def _gather_body(xf, tpad, crys, ib, rows, sem1):
    c = lax.axis_index("c")
    sid = lax.axis_index("s")
    w = sid * 2 + c
    base = w * _TPW

    def chunk(i, _):
        b0 = base + i * _TCH
        pltpu.sync_copy(tpad.at[pl.ds(b0, _TCH)], ib)
        pltpu.async_copy(xf.at[ib], rows, sem1).wait()
        pltpu.sync_copy(rows, crys.at[pl.ds(b0, _TCH)])
        return 0

    lax.fori_loop(0, _NTCH, chunk, 0)


_gather = pl.kernel(
    _gather_body,
    out_type=jax.ShapeDtypeStruct((_NPAD, _D), f32),
    mesh=_mesh,
    compiler_params=pltpu.CompilerParams(use_tc_tiling_on_sc=False),
    scratch_types=[
        pltpu.VMEM((_TCH,), i32),
        pltpu.VMEM((_TCH, _D), f32),
        pltpu.SemaphoreType.DMA,
    ],
)


def _embed_body(x_ref, emb_ref, out_ref):
    xr = x_ref[...]
    io = lax.broadcasted_iota(i32, (_BLKN, 128), 1)
    oh = (xr == io).astype(f32)
    out_ref[...] = jnp.dot(oh, emb_ref[...], preferred_element_type=f32)


_embed = pl.pallas_call(
    _embed_body,
    grid=(_GN,),
    in_specs=[
        pl.BlockSpec((_BLKN, 1), lambda i: (i, 0)),
        pl.BlockSpec((128, _D), lambda i: (0, 0)),
    ],
    out_specs=pl.BlockSpec((_BLKN, _D), lambda i: (i, 0)),
    out_shape=jax.ShapeDtypeStruct((_N, _D), f32),
)


def _nodeproj_body(xf_ref, wcat_ref, wf_ref, pil_ref, pih_ref, pjl_ref,
                   pjh_ref, fv_ref):
    acc = jnp.dot(xf_ref[...], wcat_ref[...], preferred_element_type=f32)
    pil_ref[...] = acc[:, 0:32]
    pih_ref[...] = acc[:, 32:64]
    pjl_ref[...] = acc[:, 64:96]
    pjh_ref[...] = acc[:, 96:128]
    fv_ref[...] = jnp.dot(xf_ref[...], wf_ref[...], preferred_element_type=f32)


_nodeproj = pl.pallas_call(
    _nodeproj_body,
    grid=(_GN,),
    in_specs=[
        pl.BlockSpec((_BLKN, _D), lambda i: (i, 0)),
        pl.BlockSpec((_D, 128), lambda i: (0, 0)),
        pl.BlockSpec((_D, 8), lambda i: (0, 0)),
    ],
    out_specs=[
        pl.BlockSpec((_BLKN, _H), lambda i: (i, 0)),
        pl.BlockSpec((_BLKN, _H), lambda i: (i, 0)),
        pl.BlockSpec((_BLKN, _H), lambda i: (i, 0)),
        pl.BlockSpec((_BLKN, _H), lambda i: (i, 0)),
        pl.BlockSpec((_BLKN, 8), lambda i: (i, 0)),
    ],
    out_shape=[
        jax.ShapeDtypeStruct((_N, _H), f32),
        jax.ShapeDtypeStruct((_N, _H), f32),
        jax.ShapeDtypeStruct((_N, _H), f32),
        jax.ShapeDtypeStruct((_N, _H), f32),
        jax.ShapeDtypeStruct((_N, 8), f32),
    ],
)


def _eaproj_body(ea_ref, w_ref, b_ref, eal_ref, eah_ref, eaf_ref):
    i = pl.program_id(0)
    acc = jnp.dot(ea_ref[...], w_ref[...], preferred_element_type=f32) + b_ref[...]
    rid = lax.broadcasted_iota(i32, (_BLKE, 1), 0) + i * _BLKE
    mask = rid < _E
    eal_ref[...] = jnp.where(mask, acc[:, 0:32], 0.0)
    eah_ref[...] = jnp.where(mask, acc[:, 32:64], 0.0)
    eaf_ref[...] = jnp.where(mask, acc[:, 64:72], -1e30)


_eaproj = pl.pallas_call(
    _eaproj_body,
    grid=(_GE,),
    in_specs=[
        pl.BlockSpec((_BLKE, 16), lambda i: (i, 0)),
        pl.BlockSpec((16, 128), lambda i: (0, 0)),
        pl.BlockSpec((1, 128), lambda i: (0, 0)),
    ],
    out_specs=[
        pl.BlockSpec((_BLKE, _H), lambda i: (i, 0)),
        pl.BlockSpec((_BLKE, _H), lambda i: (i, 0)),
        pl.BlockSpec((_BLKE, 8), lambda i: (i, 0)),
    ],
    out_shape=[
        jax.ShapeDtypeStruct((_EPAD, _H), f32),
        jax.ShapeDtypeStruct((_EPAD, _H), f32),
        jax.ShapeDtypeStruct((_EPAD, 8), f32),
    ],
)


def _k6a_body(aggl_ref, aggh_ref, s_ref, cnt_ref, agg_ref, ssum_ref, ssum2_ref):
    i = pl.program_id(0)
    scale = 1.0 / ((s_ref[...] + 1e-16) * jnp.maximum(cnt_ref[...], 1.0))
    a = jnp.concatenate([aggl_ref[...], aggh_ref[...]], axis=1) * scale
    agg_ref[...] = a

    @pl.when(i == 0)
    def _():
        ssum_ref[...] = jnp.zeros_like(ssum_ref)
        ssum2_ref[...] = jnp.zeros_like(ssum2_ref)

    ssum_ref[...] += jnp.sum(a, axis=0, keepdims=True)
    ssum2_ref[...] += jnp.sum(a * a, axis=0, keepdims=True)


_k6a = pl.pallas_call(
    _k6a_body,
    grid=(_GN,),
    in_specs=[
        pl.BlockSpec((_BLKN, _H), lambda i: (i, 0)),
        pl.BlockSpec((_BLKN, _H), lambda i: (i, 0)),
        pl.BlockSpec((_BLKN, 1), lambda i: (i, 0)),
        pl.BlockSpec((_BLKN, 1), lambda i: (i, 0)),
    ],
    out_specs=[
        pl.BlockSpec((_BLKN, _D), lambda i: (i, 0)),
        pl.BlockSpec((1, _D), lambda i: (0, 0)),
        pl.BlockSpec((1, _D), lambda i: (0, 0)),
    ],
    out_shape=[
        jax.ShapeDtypeStruct((_N, _D), f32),
        jax.ShapeDtypeStruct((1, _D), f32),
        jax.ShapeDtypeStruct((1, _D), f32),
    ],
)


def _k6b_body(agg_ref, xf_ref, a2_ref, b2_ref, out_ref):
    out_ref[...] = jnp.maximum(agg_ref[...] * a2_ref[...] + b2_ref[...]
                               + xf_ref[...], 0.0)


_k6b = pl.pallas_call(
    _k6b_body,
    grid=(_GN,),
    in_specs=[
        pl.BlockSpec((_BLKN, _D), lambda i: (i, 0)),
        pl.BlockSpec((_BLKN, _D), lambda i: (i, 0)),
        pl.BlockSpec((1, _D), lambda i: (0, 0)),
        pl.BlockSpec((1, _D), lambda i: (0, 0)),
    ],
    out_specs=pl.BlockSpec((_BLKN, _D), lambda i: (i, 0)),
    out_shape=jax.ShapeDtypeStruct((_N, _D), f32),
)


def _fc_body(cr_ref, w1_ref, b1_ref, dv_ref, db_ref, out_ref):
    c0 = jnp.maximum(cr_ref[...], 0.0)
    t1 = jnp.maximum(jnp.dot(c0, w1_ref[...], preferred_element_type=f32)
                     + b1_ref[...], 0.0)
    d8 = jnp.dot(t1, dv_ref[...], preferred_element_type=f32) + db_ref[...]
    d = d8[:, 0:1]
    p1 = 1.0 / (1.0 + jnp.exp(-d))
    out_ref[...] = jnp.concatenate([1.0 - p1, p1], axis=1)


_fc = pl.pallas_call(
    _fc_body,
    grid=(_GT,),
    in_specs=[
        pl.BlockSpec((_BLKT, _D), lambda i: (i, 0)),
        pl.BlockSpec((_D, _D), lambda i: (0, 0)),
        pl.BlockSpec((1, _D), lambda i: (0, 0)),
        pl.BlockSpec((_D, 8), lambda i: (0, 0)),
        pl.BlockSpec((1, 8), lambda i: (0, 0)),
    ],
    out_specs=pl.BlockSpec((_BLKT, 2), lambda i: (i, 0)),
    out_shape=jax.ShapeDtypeStruct((_NPAD, 2), f32),
)


def kernel(x, edge_index, edge_attr, target, emb, core_W, core_b, filt_W,
           filt_b, bn1_g, bn1_b, bn2_g, bn2_b, fc1_W, fc1_b, fc2_W, fc2_b):
    x = x.astype(i32)
    idx_i = edge_index[0].astype(i32)
    idx_j = edge_index[1].astype(i32)
    pad_e = _EPAD - _E
    idxi_p = jnp.concatenate([idx_i, jnp.zeros((pad_e,), i32)])
    idxj_p = jnp.concatenate([idx_j, jnp.zeros((pad_e,), i32)])
    ea_pad = jnp.concatenate([edge_attr.astype(f32),
                              jnp.zeros((pad_e, 16), f32)], axis=0)
    tpad = jnp.concatenate([target.astype(i32),
                            jnp.zeros((_NPAD - _N,), i32)])
    zn = jnp.zeros((_N,), f32)
    zn32 = jnp.zeros((_N, _H), f32)
    emb_pad = jnp.zeros((128, _D), f32).at[:100].set(emb)

    xf = _embed(x.reshape(_N, 1), emb_pad)
    cnt = None

    for l in range(3):
        cW = core_W[l]
        fW = filt_W[l]
        wcat = jnp.concatenate([cW[:_D, :_H], cW[:_D, _H:],
                                cW[_D:2 * _D, :_H], cW[_D:2 * _D, _H:]], axis=1)
        wf = jnp.zeros((_D, 8), f32).at[:, 0].set(fW[_D:2 * _D, 0])
        pil, pih, pjl, pjh, fv = _nodeproj(xf, wcat, wf)
        f_j = fv[:, 0]
        wec = jnp.zeros((16, 128), f32).at[:, :_D].set(cW[2 * _D:]) \
                                       .at[:, _D].set(fW[2 * _D:, 0])
        bec = jnp.zeros((1, 128), f32).at[0, :_D].set(core_b[l]) \
                                      .at[0, _D].set(filt_b[l][0])
        eal, eah, eaf8 = _eaproj(ea_pad, wec, bec)
        eaf = eaf8[:, 0]

        if l == 0:
            hl, hh, ex, s2, stats, cnt2 = _passA0(
                idxi_p, idxj_p, pil, pih, pjl, pjh, eal, eah, eaf, f_j, zn)
            cnt = (cnt2[0] + cnt2[1]).reshape(_N, 1)
        else:
            hl, hh, ex, s2, stats = _passA(
                idxi_p, idxj_p, pil, pih, pjl, pjh, eal, eah, eaf, f_j, zn)
        s = s2[0] + s2[1]

        sum_h = jnp.concatenate([stats[0, :, 0, :].sum(0),
                                 stats[1, :, 0, :].sum(0)])
        sum_h2 = jnp.concatenate([stats[0, :, 1, :].sum(0),
                                  stats[1, :, 1, :].sum(0)])
        hp = jnp.concatenate([pil[0] + pjl[0], pih[0] + pjh[0]])
        sum_h = sum_h - pad_e * hp
        sum_h2 = sum_h2 - pad_e * hp * hp
        mu = sum_h / _E
        var = sum_h2 / _E - mu * mu
        a_bn = bn1_g[l] * lax.rsqrt(var + 1e-5)
        b_bn = bn1_b[l] - mu * a_bn
        ab = jnp.stack([jnp.stack([a_bn[:_H], b_bn[:_H]]),
                        jnp.stack([a_bn[_H:], b_bn[_H:]])])

        aggl, aggh = _passB(hl, hh, ex, idxi_p, ab, zn32)

        agg, ssum, ssum2 = _k6a(aggl, aggh, s.reshape(_N, 1), cnt)
        mu2 = ssum / _N
        var2 = ssum2 / _N - mu2 * mu2
        a2 = bn2_g[l] * lax.rsqrt(var2 + 1e-5)
        b2 = bn2_b[l] - mu2 * a2
        xf = _k6b(agg, xf, a2.reshape(1, _D), b2.reshape(1, _D))

    crys = _gather(xf, tpad)
    dv = jnp.zeros((_D, 8), f32).at[:, 0].set(fc2_W[:, 1] - fc2_W[:, 0])
    db = jnp.zeros((1, 8), f32).at[0, 0].set(fc2_b[1] - fc2_b[0])
    out = _fc(crys, fc1_W, fc1_b.reshape(1, _D), dv, db)
    return out[:_N]


# speedup vs baseline: 7.9139x; 1.0896x over previous
import functools

import jax
import jax.numpy as jnp
from jax import lax
from jax.experimental import pallas as pl
from jax.experimental.pallas import tpu as pltpu
from jax.experimental.pallas import tpu_sc as plsc

f32 = jnp.float32
i32 = jnp.int32

_N = 50000
_E = 800000
_D = 64
_H = 32
_C = 128
_NSUB = 16
_EPAD = 802816
_EPT = _EPAD // _NSUB
_NCHUNK = _EPT // _C
_EPC = _EPAD // 2
_EPT2 = _EPC // _NSUB
_NCHUNK2 = _EPT2 // _C
_NPAD = 50176
_TPW = _NPAD // 32
_TCH = 112
_NTCH = _TPW // _TCH
_NSL = _N // _NSUB

_BLKN = 2000
_GN = _N // _BLKN
_BLKE = 4096
_GE = _EPAD // _BLKE
_BLKT = 1792
_GT = _NPAD // _BLKT

_mesh = plsc.VectorSubcoreMesh(core_axis_name="c", subcore_axis_name="s",
                               num_cores=2, num_subcores=_NSUB)


_NG = _NCHUNK // 2


_NG8 = _NCHUNK // 8


def _passA_body(do_cnt, *refs):
    if do_cnt:
        (idxi2, idxj2, pil, pih, pjl, pjh, eal, eah, eaf, fj, zn,
         hl, hh, ex, s2_out, stats, cnt2,
         iA, jA, iB, jB,
         pib0, pjb0, eab0, pib1, pjb1, eab1,
         pib2, pjb2, eab2, pib3, pjb3, eab3,
         fjb, eafb, exb, onesb, statsb, s_sh, cnt_sh,
         semA0, semA1, semA2, semA3, semW0, semW1, semW2, semW3) = refs
    else:
        (idxi2, idxj2, pil, pih, pjl, pjh, eal, eah, eaf, fj, zn,
         hl, hh, ex, s2_out, stats,
         iA, jA, iB, jB,
         pib0, pjb0, eab0, pib1, pjb1, eab1,
         pib2, pjb2, eab2, pib3, pjb3, eab3,
         fjb, eafb, exb, onesb, statsb, s_sh, cnt_sh,
         semA0, semA1, semA2, semA3, semW0, semW1, semW2, semW3) = refs
    c = lax.axis_index("c")
    sid = lax.axis_index("s")
    base = sid * _EPT
    brow = sid * _NCHUNK
    lane = lax.iota(i32, 16)

    def run(pi_r, pj_r, ea_r, h_r, f_even):
        rows = ((pib0, pjb0, eab0, semA0, semW0),
                (pib1, pjb1, eab1, semA1, semW1),
                (pib2, pjb2, eab2, semA2, semW2),
                (pib3, pjb3, eab3, semA3, semW3))

        def idxrow(k):
            k = k % 8
            return (iA.at[k % 4], jA.at[k % 4]) if k < 4 else \
                   (iB.at[k % 4], jB.at[k % 4])

        def load_quad(ii, jj, q):
            pltpu.sync_copy(idxi2.at[pl.ds(brow + q * 4, 4)], ii)
            pltpu.sync_copy(idxj2.at[pl.ds(brow + q * 4, 4)], jj)

        def with_f(k):
            return (k % 2 == 0) == f_even

        def issue(t, k):
            pib, pjb, eab, semA, _ = rows[k % 4]
            ii, jj = idxrow(k)
            b0 = base + t * _C
            pltpu.async_copy(pi_r.at[ii], pib, semA)
            pltpu.async_copy(pj_r.at[jj], pjb, semA)
            pltpu.async_copy(ea_r.at[pl.ds(b0, _C)], eab, semA)
            if with_f(k):
                pltpu.async_copy(fj.at[jj], fjb, semA)
                pltpu.async_copy(eaf.at[pl.ds(b0, _C)], eafb, semA)

        def wait_loads(t, k):
            pib, pjb, eab, semA, _ = rows[k % 4]
            ii, jj = idxrow(k)
            b0 = base + t * _C
            pltpu.make_async_copy(pi_r.at[ii], pib, semA).wait()
            pltpu.make_async_copy(pj_r.at[jj], pjb, semA).wait()
            pltpu.make_async_copy(ea_r.at[pl.ds(b0, _C)], eab, semA).wait()
            if with_f(k):
                pltpu.make_async_copy(fj.at[jj], fjb, semA).wait()
                pltpu.make_async_copy(eaf.at[pl.ds(b0, _C)], eafb, semA).wait()

        def hwrite(t, k):
            pib, _, _, _, semW = rows[k % 4]
            pltpu.async_copy(pib, h_r.at[pl.ds(base + t * _C, _C)], semW)

        def hwait(t, k):
            pib, _, _, _, semW = rows[k % 4]
            pltpu.make_async_copy(
                pib, h_r.at[pl.ds(base + t * _C, _C)], semW).wait()

        def comp(t, k, carry):
            pib, pjb, eab, _, _ = rows[k % 4]
            ii, jj = idxrow(k)

            def ebody(kk, cr):
                sh0, sh1, sq0, sq1 = cr
                h0 = pib[kk, pl.ds(0, 16)] + pjb[kk, pl.ds(0, 16)] + eab[kk, pl.ds(0, 16)]
                h1 = pib[kk, pl.ds(16, 16)] + pjb[kk, pl.ds(16, 16)] + eab[kk, pl.ds(16, 16)]
                pib[kk, pl.ds(0, 16)] = h0
                pib[kk, pl.ds(16, 16)] = h1
                return (sh0 + h0, sh1 + h1, sq0 + h0 * h0, sq1 + h1 * h1)

            zc = jnp.zeros((16,), f32)
            cs = lax.fori_loop(0, _C, ebody, (zc, zc, zc, zc), unroll=2)
            carry = tuple(g + d for g, d in zip(carry, cs))
            hwrite(t, k)
            if with_f(k):
                for m in range(_C // 16):
                    sl = pl.ds(m * 16, 16)
                    exb[sl] = jnp.exp(fjb[sl] + eafb[sl])
                b0 = base + t * _C
                pltpu.sync_copy(exb, ex.at[pl.ds(b0, _C)])
                pltpu.sync_copy(exb, s_sh.at[ii], add=True)
                if do_cnt:
                    for m in range(_C // 16):
                        gid = b0 + m * 16 + lane
                        onesb[pl.ds(m * 16, 16)] = jnp.where(gid < _E, 1.0, 0.0)
                    pltpu.sync_copy(onesb, cnt_sh.at[ii], add=True)
            return carry

        load_quad(iA, jA, 0)
        issue(0, 0)
        issue(1, 1)

        def body(b8, carry):
            t0 = 8 * b8
            load_quad(iB, jB, 2 * b8 + 1)
            for k in range(8):
                t = t0 + k
                wait_loads(t, k)
                carry = comp(t, k, carry)
                if k == 3:
                    @pl.when(b8 + 1 < _NG8)
                    def _():
                        load_quad(iA, jA, 2 * b8 + 2)
                if k < 2:
                    @pl.when(b8 > 0)
                    def _():
                        hwait(t - 2, k + 2)
                    issue(t + 2, k + 2)
                elif k < 6:
                    hwait(t - 2, k + 2)
                    issue(t + 2, k + 2)
                else:
                    @pl.when(b8 + 1 < _NG8)
                    def _():
                        hwait(t - 2, k + 2)
                        issue(t + 2, k + 2)
            return carry

        z = jnp.zeros((16,), f32)
        sh0, sh1, sq0, sq1 = lax.fori_loop(0, _NG8, body, (z, z, z, z))
        hwait(_NCHUNK - 4, 0)
        hwait(_NCHUNK - 3, 1)
        hwait(_NCHUNK - 2, 2)
        hwait(_NCHUNK - 1, 3)
        statsb[0, pl.ds(0, 16)] = sh0
        statsb[0, pl.ds(16, 16)] = sh1
        statsb[1, pl.ds(0, 16)] = sq0
        statsb[1, pl.ds(16, 16)] = sq1
        pltpu.sync_copy(statsb, stats.at[c, sid])

    @pl.when(sid == 0)
    def _():
        pltpu.sync_copy(zn, s_sh)
        if do_cnt:
            pltpu.sync_copy(zn, cnt_sh)

    plsc.subcore_barrier()

    @pl.when(c == 0)
    def _():
        run(pil, pjl, eal, hl, True)

    @pl.when(c == 1)
    def _():
        run(pih, pjh, eah, hh, False)

    plsc.subcore_barrier()

    @pl.when(sid == 0)
    def _():
        pltpu.sync_copy(s_sh, s2_out.at[c])
        if do_cnt:
            pltpu.sync_copy(cnt_sh, cnt2.at[c])


_passA_outs = [
    jax.ShapeDtypeStruct((_EPAD, _H), f32),
    jax.ShapeDtypeStruct((_EPAD, _H), f32),
    jax.ShapeDtypeStruct((_EPAD,), f32),
    jax.ShapeDtypeStruct((2, _N), f32),
    jax.ShapeDtypeStruct((2, _NSUB, 2, _H), f32),
]
_passA_scratch = [
        pltpu.VMEM((4, _C), i32),
        pltpu.VMEM((4, _C), i32),
        pltpu.VMEM((4, _C), i32),
        pltpu.VMEM((4, _C), i32),
        pltpu.VMEM((_C, _H), f32),
        pltpu.VMEM((_C, _H), f32),
        pltpu.VMEM((_C, _H), f32),
        pltpu.VMEM((_C, _H), f32),
        pltpu.VMEM((_C, _H), f32),
        pltpu.VMEM((_C, _H), f32),
        pltpu.VMEM((_C, _H), f32),
        pltpu.VMEM((_C, _H), f32),
        pltpu.VMEM((_C, _H), f32),
        pltpu.VMEM((_C, _H), f32),
        pltpu.VMEM((_C, _H), f32),
        pltpu.VMEM((_C, _H), f32),
        pltpu.VMEM((_C,), f32),
        pltpu.VMEM((_C,), f32),
        pltpu.VMEM((_C,), f32),
        pltpu.VMEM((_C,), f32),
        pltpu.VMEM((2, _H), f32),
        pltpu.VMEM_SHARED((_N,), f32),
        pltpu.VMEM_SHARED((_N,), f32),
        pltpu.SemaphoreType.DMA,
        pltpu.SemaphoreType.DMA,
        pltpu.SemaphoreType.DMA,
        pltpu.SemaphoreType.DMA,
        pltpu.SemaphoreType.DMA,
        pltpu.SemaphoreType.DMA,
        pltpu.SemaphoreType.DMA,
        pltpu.SemaphoreType.DMA,
]

_passA = pl.kernel(
    functools.partial(_passA_body, False),
    out_type=_passA_outs,
    mesh=_mesh,
    compiler_params=pltpu.CompilerParams(use_tc_tiling_on_sc=False),
    scratch_types=_passA_scratch,
)

_passA0 = pl.kernel(
    functools.partial(_passA_body, True),
    out_type=_passA_outs + [jax.ShapeDtypeStruct((2, _N), f32)],
    mesh=_mesh,
    compiler_params=pltpu.CompilerParams(use_tc_tiling_on_sc=False),
    scratch_types=_passA_scratch,
)


def _passB_body(hl, hh, ex, idxi, ab, zn32, aggl, aggh,
                hb0, exb0, ib0, hb1, exb1, ib1, abv, acc_sh, semB0, semB1):
    c = lax.axis_index("c")
    sid = lax.axis_index("s")
    base = sid * _EPT
    pltpu.sync_copy(ab.at[c], abv)

    @pl.when(sid == 0)
    def _():
        pltpu.sync_copy(zn32, acc_sh)

    plsc.subcore_barrier()
    a0 = abv[0, pl.ds(0, 16)]
    a1 = abv[0, pl.ds(16, 16)]
    b0v = abv[1, pl.ds(0, 16)]
    b1v = abv[1, pl.ds(16, 16)]

    def half(h_r, agg_r):
        bufs = ((hb0, exb0, ib0, semB0), (hb1, exb1, ib1, semB1))

        def issue(cidx, b):
            hb, exb, ib, sem = bufs[b]
            b0 = base + cidx * _C
            pltpu.async_copy(h_r.at[pl.ds(b0, _C)], hb, sem)
            pltpu.async_copy(ex.at[pl.ds(b0, _C)], exb, sem)
            pltpu.async_copy(idxi.at[pl.ds(b0, _C)], ib, sem)

        def wait_loads(cidx, b):
            hb, exb, ib, sem = bufs[b]
            b0 = base + cidx * _C
            pltpu.make_async_copy(h_r.at[pl.ds(b0, _C)], hb, sem).wait()
            pltpu.make_async_copy(ex.at[pl.ds(b0, _C)], exb, sem).wait()
            pltpu.make_async_copy(idxi.at[pl.ds(b0, _C)], ib, sem).wait()

        def comp(b):
            hb, exb, ib, sem = bufs[b]

            def gbody(g, __):
                ex16 = exb[pl.ds(g * 16, 16)]
                for j in range(16):
                    k = g * 16 + j
                    sc = ex16[j]
                    h0 = hb[k, pl.ds(0, 16)]
                    h1 = hb[k, pl.ds(16, 16)]
                    hb[k, pl.ds(0, 16)] = jnp.maximum(h0 * a0 + b0v, 0.0) * sc
                    hb[k, pl.ds(16, 16)] = jnp.maximum(h1 * a1 + b1v, 0.0) * sc
                return 0

            lax.fori_loop(0, _C // 16, gbody, 0)
            pltpu.sync_copy(hb, acc_sh.at[ib], add=True)

        issue(0, 0)

        def body(gg, _):
            c0 = 2 * gg
            issue(c0 + 1, 1)
            wait_loads(c0, 0)
            comp(0)

            @pl.when(gg + 1 < _NG)
            def _():
                issue(c0 + 2, 0)

            wait_loads(c0 + 1, 1)
            comp(1)
            return 0

        lax.fori_loop(0, _NG, body, 0)
        plsc.subcore_barrier()
        pltpu.sync_copy(acc_sh.at[pl.ds(sid * _NSL, _NSL)],
                        agg_r.at[pl.ds(sid * _NSL, _NSL)])

    @pl.when(c == 0)
    def _():
        half(hl, aggl)

    @pl.when(c == 1)
    def _():
        half(hh, aggh)


_passB = pl.kernel(
    _passB_body,
    out_type=[
        jax.ShapeDtypeStruct((_N, _H), f32),
        jax.ShapeDtypeStruct((_N, _H), f32),
    ],
    mesh=_mesh,
    compiler_params=pltpu.CompilerParams(use_tc_tiling_on_sc=False),
    scratch_types=[
        pltpu.VMEM((_C, _H), f32),
        pltpu.VMEM((_C,), f32),
        pltpu.VMEM((_C,), i32),
        pltpu.VMEM((_C, _H), f32),
        pltpu.VMEM((_C,), f32),
        pltpu.VMEM((_C,), i32),
        pltpu.VMEM((2, _H), f32),
        pltpu.VMEM_SHARED((_N, _H), f32),
        pltpu.SemaphoreType.DMA,
        pltpu.SemaphoreType.DMA,
    ],
)


def _gather_body(xf, tpad, crys, ib, rows, sem1):
    c = lax.axis_index("c")
    sid = lax.axis_index("s")
    w = sid * 2 + c
    base = w * _TPW

    def chunk(i, _):
        b0 = base + i * _TCH
        pltpu.sync_copy(tpad.at[pl.ds(b0, _TCH)], ib)
        pltpu.async_copy(xf.at[ib], rows, sem1).wait()
        pltpu.sync_copy(rows, crys.at[pl.ds(b0, _TCH)])
        return 0

    lax.fori_loop(0, _NTCH, chunk, 0)


_gather = pl.kernel(
    _gather_body,
    out_type=jax.ShapeDtypeStruct((_NPAD, _D), f32),
    mesh=_mesh,
    compiler_params=pltpu.CompilerParams(use_tc_tiling_on_sc=False),
    scratch_types=[
        pltpu.VMEM((_TCH,), i32),
        pltpu.VMEM((_TCH, _D), f32),
        pltpu.SemaphoreType.DMA,
    ],
)


def _embed_body(x_ref, emb_ref, out_ref):
    xr = x_ref[...]
    io = lax.broadcasted_iota(i32, (_BLKN, 128), 1)
    oh = (xr == io).astype(f32)
    out_ref[...] = jnp.dot(oh, emb_ref[...], preferred_element_type=f32)


_embed = pl.pallas_call(
    _embed_body,
    grid=(_GN,),
    in_specs=[
        pl.BlockSpec((_BLKN, 1), lambda i: (i, 0)),
        pl.BlockSpec((128, _D), lambda i: (0, 0)),
    ],
    out_specs=pl.BlockSpec((_BLKN, _D), lambda i: (i, 0)),
    out_shape=jax.ShapeDtypeStruct((_N, _D), f32),
)


def _nodeproj_body(xf_ref, wcat_ref, wf_ref, pil_ref, pih_ref, pjl_ref,
                   pjh_ref, fv_ref):
    acc = jnp.dot(xf_ref[...], wcat_ref[...], preferred_element_type=f32)
    pil_ref[...] = acc[:, 0:32]
    pih_ref[...] = acc[:, 32:64]
    pjl_ref[...] = acc[:, 64:96]
    pjh_ref[...] = acc[:, 96:128]
    fv_ref[...] = jnp.dot(xf_ref[...], wf_ref[...], preferred_element_type=f32)


_nodeproj = pl.pallas_call(
    _nodeproj_body,
    grid=(_GN,),
    in_specs=[
        pl.BlockSpec((_BLKN, _D), lambda i: (i, 0)),
        pl.BlockSpec((_D, 128), lambda i: (0, 0)),
        pl.BlockSpec((_D, 8), lambda i: (0, 0)),
    ],
    out_specs=[
        pl.BlockSpec((_BLKN, _H), lambda i: (i, 0)),
        pl.BlockSpec((_BLKN, _H), lambda i: (i, 0)),
        pl.BlockSpec((_BLKN, _H), lambda i: (i, 0)),
        pl.BlockSpec((_BLKN, _H), lambda i: (i, 0)),
        pl.BlockSpec((_BLKN, 8), lambda i: (i, 0)),
    ],
    out_shape=[
        jax.ShapeDtypeStruct((_N, _H), f32),
        jax.ShapeDtypeStruct((_N, _H), f32),
        jax.ShapeDtypeStruct((_N, _H), f32),
        jax.ShapeDtypeStruct((_N, _H), f32),
        jax.ShapeDtypeStruct((_N, 8), f32),
    ],
)


def _eaproj_body(ea_ref, w_ref, b_ref, eal_ref, eah_ref, eaf_ref):
    i = pl.program_id(0)
    acc = jnp.dot(ea_ref[...], w_ref[...], preferred_element_type=f32) + b_ref[...]
    rid = lax.broadcasted_iota(i32, (_BLKE, 1), 0) + i * _BLKE
    mask = rid < _E
    eal_ref[...] = jnp.where(mask, acc[:, 0:32], 0.0)
    eah_ref[...] = jnp.where(mask, acc[:, 32:64], 0.0)
    eaf_ref[...] = jnp.where(mask, acc[:, 64:72], -1e30)


_eaproj = pl.pallas_call(
    _eaproj_body,
    grid=(_GE,),
    in_specs=[
        pl.BlockSpec((_BLKE, 16), lambda i: (i, 0)),
        pl.BlockSpec((16, 128), lambda i: (0, 0)),
        pl.BlockSpec((1, 128), lambda i: (0, 0)),
    ],
    out_specs=[
        pl.BlockSpec((_BLKE, _H), lambda i: (i, 0)),
        pl.BlockSpec((_BLKE, _H), lambda i: (i, 0)),
        pl.BlockSpec((_BLKE, 8), lambda i: (i, 0)),
    ],
    out_shape=[
        jax.ShapeDtypeStruct((_EPAD, _H), f32),
        jax.ShapeDtypeStruct((_EPAD, _H), f32),
        jax.ShapeDtypeStruct((_EPAD, 8), f32),
    ],
)


def _k6a_body(aggl_ref, aggh_ref, s_ref, cnt_ref, agg_ref, ssum_ref, ssum2_ref):
    i = pl.program_id(0)
    scale = 1.0 / ((s_ref[...] + 1e-16) * jnp.maximum(cnt_ref[...], 1.0))
    a = jnp.concatenate([aggl_ref[...], aggh_ref[...]], axis=1) * scale
    agg_ref[...] = a

    @pl.when(i == 0)
    def _():
        ssum_ref[...] = jnp.zeros_like(ssum_ref)
        ssum2_ref[...] = jnp.zeros_like(ssum2_ref)

    ssum_ref[...] += jnp.sum(a, axis=0, keepdims=True)
    ssum2_ref[...] += jnp.sum(a * a, axis=0, keepdims=True)


_k6a = pl.pallas_call(
    _k6a_body,
    grid=(_GN,),
    in_specs=[
        pl.BlockSpec((_BLKN, _H), lambda i: (i, 0)),
        pl.BlockSpec((_BLKN, _H), lambda i: (i, 0)),
        pl.BlockSpec((_BLKN, 1), lambda i: (i, 0)),
        pl.BlockSpec((_BLKN, 1), lambda i: (i, 0)),
    ],
    out_specs=[
        pl.BlockSpec((_BLKN, _D), lambda i: (i, 0)),
        pl.BlockSpec((1, _D), lambda i: (0, 0)),
        pl.BlockSpec((1, _D), lambda i: (0, 0)),
    ],
    out_shape=[
        jax.ShapeDtypeStruct((_N, _D), f32),
        jax.ShapeDtypeStruct((1, _D), f32),
        jax.ShapeDtypeStruct((1, _D), f32),
    ],
)


def _k6b_body(agg_ref, xf_ref, a2_ref, b2_ref, out_ref):
    out_ref[...] = jnp.maximum(agg_ref[...] * a2_ref[...] + b2_ref[...]
                               + xf_ref[...], 0.0)


_k6b = pl.pallas_call(
    _k6b_body,
    grid=(_GN,),
    in_specs=[
        pl.BlockSpec((_BLKN, _D), lambda i: (i, 0)),
        pl.BlockSpec((_BLKN, _D), lambda i: (i, 0)),
        pl.BlockSpec((1, _D), lambda i: (0, 0)),
        pl.BlockSpec((1, _D), lambda i: (0, 0)),
    ],
    out_specs=pl.BlockSpec((_BLKN, _D), lambda i: (i, 0)),
    out_shape=jax.ShapeDtypeStruct((_N, _D), f32),
)


def _fc_body(cr_ref, w1_ref, b1_ref, dv_ref, db_ref, out_ref):
    c0 = jnp.maximum(cr_ref[...], 0.0)
    t1 = jnp.maximum(jnp.dot(c0, w1_ref[...], preferred_element_type=f32)
                     + b1_ref[...], 0.0)
    d8 = jnp.dot(t1, dv_ref[...], preferred_element_type=f32) + db_ref[...]
    d = d8[:, 0:1]
    p1 = 1.0 / (1.0 + jnp.exp(-d))
    out_ref[...] = jnp.concatenate([1.0 - p1, p1], axis=1)


_fc = pl.pallas_call(
    _fc_body,
    grid=(_GT,),
    in_specs=[
        pl.BlockSpec((_BLKT, _D), lambda i: (i, 0)),
        pl.BlockSpec((_D, _D), lambda i: (0, 0)),
        pl.BlockSpec((1, _D), lambda i: (0, 0)),
        pl.BlockSpec((_D, 8), lambda i: (0, 0)),
        pl.BlockSpec((1, 8), lambda i: (0, 0)),
    ],
    out_specs=pl.BlockSpec((_BLKT, 2), lambda i: (i, 0)),
    out_shape=jax.ShapeDtypeStruct((_NPAD, 2), f32),
)


def kernel(x, edge_index, edge_attr, target, emb, core_W, core_b, filt_W,
           filt_b, bn1_g, bn1_b, bn2_g, bn2_b, fc1_W, fc1_b, fc2_W, fc2_b):
    x = x.astype(i32)
    idx_i = edge_index[0].astype(i32)
    idx_j = edge_index[1].astype(i32)
    pad_e = _EPAD - _E
    idxi_p = jnp.concatenate([idx_i, jnp.zeros((pad_e,), i32)])
    idxj_p = jnp.concatenate([idx_j, jnp.zeros((pad_e,), i32)])
    ea_pad = jnp.concatenate([edge_attr.astype(f32),
                              jnp.zeros((pad_e, 16), f32)], axis=0)
    tpad = jnp.concatenate([target.astype(i32),
                            jnp.zeros((_NPAD - _N,), i32)])
    zn = jnp.zeros((_N,), f32)
    zn32 = jnp.zeros((_N, _H), f32)
    emb_pad = jnp.zeros((128, _D), f32).at[:100].set(emb)

    idxi2 = idxi_p.reshape(_EPAD // _C, _C)
    idxj2 = idxj_p.reshape(_EPAD // _C, _C)
    xf = _embed(x.reshape(_N, 1), emb_pad)
    cnt = None

    for l in range(3):
        cW = core_W[l]
        fW = filt_W[l]
        wcat = jnp.concatenate([cW[:_D, :_H], cW[:_D, _H:],
                                cW[_D:2 * _D, :_H], cW[_D:2 * _D, _H:]], axis=1)
        wf = jnp.zeros((_D, 8), f32).at[:, 0].set(fW[_D:2 * _D, 0])
        pil, pih, pjl, pjh, fv = _nodeproj(xf, wcat, wf)
        f_j = fv[:, 0]
        wec = jnp.zeros((16, 128), f32).at[:, :_D].set(cW[2 * _D:]) \
                                       .at[:, _D].set(fW[2 * _D:, 0])
        bec = jnp.zeros((1, 128), f32).at[0, :_D].set(core_b[l]) \
                                      .at[0, _D].set(filt_b[l][0])
        eal, eah, eaf8 = _eaproj(ea_pad, wec, bec)
        eaf = eaf8[:, 0]

        if l == 0:
            hl, hh, ex, s2, stats, cnt2 = _passA0(
                idxi2, idxj2, pil, pih, pjl, pjh, eal, eah, eaf, f_j, zn)
            cnt = (cnt2[0] + cnt2[1]).reshape(_N, 1)
        else:
            hl, hh, ex, s2, stats = _passA(
                idxi2, idxj2, pil, pih, pjl, pjh, eal, eah, eaf, f_j, zn)
        s = s2[0] + s2[1]

        sum_h = jnp.concatenate([stats[0, :, 0, :].sum(0),
                                 stats[1, :, 0, :].sum(0)])
        sum_h2 = jnp.concatenate([stats[0, :, 1, :].sum(0),
                                  stats[1, :, 1, :].sum(0)])
        hp = jnp.concatenate([pil[0] + pjl[0], pih[0] + pjh[0]])
        sum_h = sum_h - pad_e * hp
        sum_h2 = sum_h2 - pad_e * hp * hp
        mu = sum_h / _E
        var = sum_h2 / _E - mu * mu
        a_bn = bn1_g[l] * lax.rsqrt(var + 1e-5)
        b_bn = bn1_b[l] - mu * a_bn
        ab = jnp.stack([jnp.stack([a_bn[:_H], b_bn[:_H]]),
                        jnp.stack([a_bn[_H:], b_bn[_H:]])])

        aggl, aggh = _passB(hl, hh, ex, idxi_p, ab, zn32)

        agg, ssum, ssum2 = _k6a(aggl, aggh, s.reshape(_N, 1), cnt)
        mu2 = ssum / _N
        var2 = ssum2 / _N - mu2 * mu2
        a2 = bn2_g[l] * lax.rsqrt(var2 + 1e-5)
        b2 = bn2_b[l] - mu2 * a2
        xf = _k6b(agg, xf, a2.reshape(1, _D), b2.reshape(1, _D))

    crys = _gather(xf, tpad)
    dv = jnp.zeros((_D, 8), f32).at[:, 0].set(fc2_W[:, 1] - fc2_W[:, 0])
    db = jnp.zeros((1, 8), f32).at[0, 0].set(fc2_b[1] - fc2_b[0])
    out = _fc(crys, fc1_W, fc1_b.reshape(1, _D), dv, db)
    return out[:_N]
